# Initial kernel scaffold; baseline (speedup 1.0000x reference)
import jax
import jax.numpy as jnp
from jax.experimental import pallas as pl


def kernel(x_author, x_paper, edge_index_writes, edge_index_cites, W_in_author, W_in_paper, Wl_w0, bl_w0, Wr_w0, Wl_c0, bl_c0, Wr_c0, Wl_w1, bl_w1, Wr_w1, Wl_c1, bl_c1, Wr_c1):
    raise NotImplementedError("write your pallas kernel here")


# speedup vs baseline: 6.0040x; 6.0040x over previous
import functools

import jax
import jax.numpy as jnp
from jax import lax
from jax.experimental import pallas as pl
from jax.experimental.pallas import tpu as pltpu
from jax.experimental.pallas import tpu_sc as plsc

N_NODE = 10000
D_IN = 256
D_H = 128
DH2 = D_H // 2
E = 320000

NC = 2
NS = 16
CH = 128
CPT = 160
EP = NS * CPT * CH
NCH_REAL = E // CH
CPT_LAST = NCH_REAL - (NS - 1) * CPT
R = 10240
RPT = R // NS
WB = RPT // CH
BM = 1024
GRID = R // BM

_mesh = plsc.VectorSubcoreMesh(core_axis_name="c", subcore_axis_name="s")
_sc_params = pltpu.CompilerParams(use_tc_tiling_on_sc=False)


def _seg_pass(h_hbm, src_hbm, dst_hbm, osum, ocnt, count_core,
              idx_s, idx_d, rows, ones_v, zbuf, zc, cbuf, acc, cnt, c, s):
    rbase = s * RPT
    with_counts = ocnt is not None

    for j in range(WB):
        pltpu.sync_copy(zbuf, acc.at[pl.ds(rbase + j * CH, CH)])
    if with_counts:
        for j in range(WB):
            pltpu.sync_copy(zc, cnt.at[pl.ds(rbase + j * CH, CH)])
    plsc.subcore_barrier()

    pltpu.sync_copy(src_hbm.at[s], idx_s)
    pltpu.sync_copy(dst_hbm.at[s], idx_d)

    h_half = h_hbm.at[c]
    trip = jnp.where(s == NS - 1, CPT_LAST, CPT)

    if with_counts:
        @pl.loop(0, trip)
        def _(j):
            pltpu.sync_copy(h_half.at[idx_s.at[j]], rows)
            pltpu.sync_copy(rows, acc.at[idx_d.at[j]], add=True)
            @pl.when(c == count_core)
            def _():
                pltpu.sync_copy(ones_v, cnt.at[idx_d.at[j]], add=True)
    else:
        @pl.loop(0, trip)
        def _(j):
            pltpu.sync_copy(h_half.at[idx_s.at[j]], rows)
            pltpu.sync_copy(rows, acc.at[idx_d.at[j]], add=True)

    plsc.subcore_barrier()

    for j in range(WB):
        pltpu.sync_copy(acc.at[pl.ds(rbase + j * CH, CH)], rows)
        pltpu.sync_copy(rows, osum.at[c].at[pl.ds(rbase + j * CH, CH)])
    if with_counts:
        @pl.when(c == count_core)
        def _():
            for j in range(WB):
                pltpu.sync_copy(cnt.at[pl.ds(rbase + j * CH, CH)], cbuf)
                pltpu.sync_copy(cbuf, ocnt.at[pl.ds(rbase + j * CH, CH)])
    plsc.subcore_barrier()


@functools.partial(
    pl.kernel,
    out_type=[
        jax.ShapeDtypeStruct((NC, R, DH2), jnp.float32),
        jax.ShapeDtypeStruct((R,), jnp.float32),
        jax.ShapeDtypeStruct((NC, R, DH2), jnp.float32),
        jax.ShapeDtypeStruct((R,), jnp.float32),
    ],
    mesh=_mesh,
    scratch_types=[
        pltpu.VMEM((CPT, CH), jnp.int32),
        pltpu.VMEM((CPT, CH), jnp.int32),
        pltpu.VMEM((CH, DH2), jnp.float32),
        pltpu.VMEM((CH,), jnp.float32),
        pltpu.VMEM((CH, DH2), jnp.float32),
        pltpu.VMEM((CH,), jnp.float32),
        pltpu.VMEM((CH,), jnp.float32),
        pltpu.VMEM_SHARED((R, DH2), jnp.float32),
        pltpu.VMEM_SHARED((R,), jnp.float32),
    ],
    compiler_params=_sc_params,
)
def _sc_seg_both(ha, hp, srcw, dstw, srcc, dstc, z2, z1, o1,
                 osw, ocw, osc, occ,
                 idx_s, idx_d, rows, ones_v, zbuf, zc, cbuf, acc, cnt):
    c = lax.axis_index("c")
    s = lax.axis_index("s")
    pltpu.sync_copy(z2, zbuf)
    pltpu.sync_copy(z1, zc)
    pltpu.sync_copy(o1, ones_v)
    _seg_pass(ha, srcw, dstw, osw, ocw, 0,
              idx_s, idx_d, rows, ones_v, zbuf, zc, cbuf, acc, cnt, c, s)
    _seg_pass(hp, srcc, dstc, osc, occ, 1,
              idx_s, idx_d, rows, ones_v, zbuf, zc, cbuf, acc, cnt, c, s)


@functools.partial(
    pl.kernel,
    out_type=[jax.ShapeDtypeStruct((NC, R, DH2), jnp.float32)],
    mesh=_mesh,
    scratch_types=[
        pltpu.VMEM((CPT, CH), jnp.int32),
        pltpu.VMEM((CPT, CH), jnp.int32),
        pltpu.VMEM((CH, DH2), jnp.float32),
        pltpu.VMEM((CH, DH2), jnp.float32),
        pltpu.VMEM_SHARED((R, DH2), jnp.float32),
    ],
    compiler_params=_sc_params,
)
def _sc_seg_cites(hp, srcc, dstc, z2, osc,
                  idx_s, idx_d, rows, zbuf, acc):
    c = lax.axis_index("c")
    s = lax.axis_index("s")
    pltpu.sync_copy(z2, zbuf)
    _seg_pass(hp, srcc, dstc, osc, None, None,
              idx_s, idx_d, rows, None, zbuf, None, None, acc, None, c, s)


def _proj_body(xa_ref, xp_ref, wa_ref, wp_ref, haf_ref, has_ref, hps_ref):
    ha = jnp.dot(xa_ref[...], wa_ref[...], preferred_element_type=jnp.float32)
    hp = jnp.dot(xp_ref[...], wp_ref[...], preferred_element_type=jnp.float32)
    haf_ref[...] = ha
    has_ref[0, ...] = ha[:, :DH2]
    has_ref[1, ...] = ha[:, DH2:]
    hps_ref[0, ...] = hp[:, :DH2]
    hps_ref[1, ...] = hp[:, DH2:]


def _project(xa, xp, wa, wp):
    return pl.pallas_call(
        _proj_body,
        grid=(GRID,),
        in_specs=[
            pl.BlockSpec((BM, D_IN), lambda i: (i, 0)),
            pl.BlockSpec((BM, D_IN), lambda i: (i, 0)),
            pl.BlockSpec((D_IN, D_H), lambda i: (0, 0)),
            pl.BlockSpec((D_IN, D_H), lambda i: (0, 0)),
        ],
        out_specs=[
            pl.BlockSpec((BM, D_H), lambda i: (i, 0)),
            pl.BlockSpec((NC, BM, DH2), lambda i: (0, i, 0)),
            pl.BlockSpec((NC, BM, DH2), lambda i: (0, i, 0)),
        ],
        out_shape=[
            jax.ShapeDtypeStruct((R, D_H), jnp.float32),
            jax.ShapeDtypeStruct((NC, R, DH2), jnp.float32),
            jax.ShapeDtypeStruct((NC, R, DH2), jnp.float32),
        ],
    )(xa, xp, wa, wp)


def _inv_counts_body(cw_ref, cc_ref, iw_ref, ic_ref):
    r0 = lax.broadcasted_iota(jnp.int32, (D_H, DH2), 0)
    r1 = lax.broadcasted_iota(jnp.int32, (D_H, DH2), 1)
    sel = (r0 == r1).astype(jnp.float32)
    for src, dst in ((cw_ref, iw_ref), (cc_ref, ic_ref)):
        inv8 = 1.0 / jnp.maximum(src[...], 1.0)
        for k in range(8):
            rb = jnp.broadcast_to(inv8[k:k + 1, :], (D_H, D_H))
            part = lax.dot_general(rb, sel, (((0,), (0,)), ((), ())),
                                   preferred_element_type=jnp.float32)
            dst[pl.ds(k * D_H, D_H), :] = part


def _inv_counts(cw, cc):
    return pl.pallas_call(
        _inv_counts_body,
        grid=(GRID,),
        in_specs=[
            pl.BlockSpec((8, D_H), lambda i: (i, 0)),
            pl.BlockSpec((8, D_H), lambda i: (i, 0)),
        ],
        out_specs=[
            pl.BlockSpec((BM, DH2), lambda i: (i, 0)),
            pl.BlockSpec((BM, DH2), lambda i: (i, 0)),
        ],
        out_shape=[
            jax.ShapeDtypeStruct((R, DH2), jnp.float32),
            jax.ShapeDtypeStruct((R, DH2), jnp.float32),
        ],
    )(cw.reshape(R // D_H, D_H), cc.reshape(R // D_H, D_H))


def _combine_body(sw_ref, iw_ref, sc_ref, ic_ref, hp_ref,
                  wlw_ref, wlc_ref, wrw_ref, wrc_ref, blw_ref, blc_ref,
                  out_ref, *, emit):
    iw = iw_ref[...]
    ic = ic_ref[...]
    wlw = wlw_ref[...]
    wlc = wlc_ref[...]
    wr = wrw_ref[...] + wrc_ref[...]
    dot = functools.partial(jnp.dot, preferred_element_type=jnp.float32)
    acc = dot(sw_ref[0] * iw, wlw[:DH2])
    acc = acc + dot(sw_ref[1] * iw, wlw[DH2:])
    acc = acc + dot(sc_ref[0] * ic, wlc[:DH2])
    acc = acc + dot(sc_ref[1] * ic, wlc[DH2:])
    acc = acc + dot(hp_ref[0], wr[:DH2])
    acc = acc + dot(hp_ref[1], wr[DH2:])
    acc = acc + blw_ref[...] + blc_ref[...]
    res = jnp.maximum(acc * 0.5, 0.0)
    if emit == "full":
        out_ref[...] = res
    else:
        out_ref[0, ...] = res[:, :DH2]
        out_ref[1, ...] = res[:, DH2:]


def _combine(sw, iw, sc, ic, hp, wlw, wlc, wrw, wrc, blw, blc, emit):
    if emit == "full":
        out_spec = pl.BlockSpec((BM, D_H), lambda i: (i, 0))
        out_shape = jax.ShapeDtypeStruct((R, D_H), jnp.float32)
    else:
        out_spec = pl.BlockSpec((NC, BM, DH2), lambda i: (0, i, 0))
        out_shape = jax.ShapeDtypeStruct((NC, R, DH2), jnp.float32)
    return pl.pallas_call(
        functools.partial(_combine_body, emit=emit),
        grid=(GRID,),
        in_specs=[
            pl.BlockSpec((NC, BM, DH2), lambda i: (0, i, 0)),
            pl.BlockSpec((BM, DH2), lambda i: (i, 0)),
            pl.BlockSpec((NC, BM, DH2), lambda i: (0, i, 0)),
            pl.BlockSpec((BM, DH2), lambda i: (i, 0)),
            pl.BlockSpec((NC, BM, DH2), lambda i: (0, i, 0)),
            pl.BlockSpec((D_H, D_H), lambda i: (0, 0)),
            pl.BlockSpec((D_H, D_H), lambda i: (0, 0)),
            pl.BlockSpec((D_H, D_H), lambda i: (0, 0)),
            pl.BlockSpec((D_H, D_H), lambda i: (0, 0)),
            pl.BlockSpec((1, D_H), lambda i: (0, 0)),
            pl.BlockSpec((1, D_H), lambda i: (0, 0)),
        ],
        out_specs=out_spec,
        out_shape=out_shape,
    )(sw, iw, sc, ic, hp, wlw, wlc, wrw, wrc, blw, blc)


def _prep_edges(ei):
    pad = EP - E
    src = jnp.concatenate([ei[0], jnp.zeros((pad,), jnp.int32)])
    dst = jnp.concatenate([ei[1], jnp.zeros((pad,), jnp.int32)])
    return src.reshape(NS, CPT, CH), dst.reshape(NS, CPT, CH)


def kernel(x_author, x_paper, edge_index_writes, edge_index_cites,
           W_in_author, W_in_paper,
           Wl_w0, bl_w0, Wr_w0, Wl_c0, bl_c0, Wr_c0,
           Wl_w1, bl_w1, Wr_w1, Wl_c1, bl_c1, Wr_c1):
    pad_rows = R - N_NODE
    xa = jnp.pad(x_author, ((0, pad_rows), (0, 0)))
    xp = jnp.pad(x_paper, ((0, pad_rows), (0, 0)))
    srcw, dstw = _prep_edges(edge_index_writes)
    srcc, dstc = _prep_edges(edge_index_cites)
    z2 = jnp.zeros((CH, DH2), jnp.float32)
    z1 = jnp.zeros((CH,), jnp.float32)
    o1 = jnp.ones((CH,), jnp.float32)

    haf, has, hp0s = _project(xa, xp, W_in_author, W_in_paper)

    sw, cw, sc0, cc = _sc_seg_both(has, hp0s, srcw, dstw, srcc, dstc,
                                   z2, z1, o1)
    iw, ic = _inv_counts(cw, cc)

    blw0 = bl_w0.reshape(1, D_H)
    blc0 = bl_c0.reshape(1, D_H)
    hp1s = _combine(sw, iw, sc0, ic, hp0s,
                    Wl_w0, Wl_c0, Wr_w0, Wr_c0, blw0, blc0, "split")

    (sc1,) = _sc_seg_cites(hp1s, srcc, dstc, z2)

    blw1 = bl_w1.reshape(1, D_H)
    blc1 = bl_c1.reshape(1, D_H)
    hp2 = _combine(sw, iw, sc1, ic, hp1s,
                   Wl_w1, Wl_c1, Wr_w1, Wr_c1, blw1, blc1, "full")

    return (haf[:N_NODE], hp2[:N_NODE])


# speedup vs baseline: 9.3086x; 1.5504x over previous
import functools

import jax
import jax.numpy as jnp
from jax import lax
from jax.experimental import pallas as pl
from jax.experimental.pallas import tpu as pltpu
from jax.experimental.pallas import tpu_sc as plsc

N_NODE = 10000
D_IN = 256
D_H = 128
DH2 = D_H // 2
E = 320000

NC = 2
NS = 16
CH = 128
CPT = 160
EP = NS * CPT * CH
NCH_REAL = E // CH
CPT_LAST = NCH_REAL - (NS - 1) * CPT
R = 10240
RPT = R // NS
WB = RPT // CH
BM = 1024
GRID = R // BM

_mesh = plsc.VectorSubcoreMesh(core_axis_name="c", subcore_axis_name="s")
_sc_params = pltpu.CompilerParams(use_tc_tiling_on_sc=False)


def _seg_pass(h_hbm, src_hbm, dst_hbm, osum, ocnt, count_core,
              idx_s, idx_d, rows0, rows1, sem0, sem1,
              ones_v, zbuf, zc, cbuf, acc, cnt, c, s):
    rbase = s * RPT
    with_counts = ocnt is not None

    for j in range(WB):
        pltpu.sync_copy(zbuf, acc.at[pl.ds(rbase + j * CH, CH)])
    if with_counts:
        for j in range(WB):
            pltpu.sync_copy(zc, cnt.at[pl.ds(rbase + j * CH, CH)])
    plsc.subcore_barrier()

    pltpu.sync_copy(src_hbm.at[s], idx_s)
    pltpu.sync_copy(dst_hbm.at[s], idx_d)

    h_half = h_hbm.at[c]
    trip = jnp.where(s == NS - 1, CPT_LAST, CPT)

    def start(j, rows, sem):
        pltpu.async_copy(h_half.at[idx_s.at[j]], rows, sem)

    def wait(j, rows, sem):
        pltpu.make_async_copy(h_half.at[idx_s.at[j]], rows, sem).wait()

    def consume(j, rows):
        pltpu.sync_copy(rows, acc.at[idx_d.at[j]], add=True)
        if with_counts:
            @pl.when(c == count_core)
            def _():
                pltpu.sync_copy(ones_v, cnt.at[idx_d.at[j]], add=True)

    start(0, rows0, sem0)
    start(1, rows1, sem1)

    @pl.loop(0, trip - 2, step=2)
    def _(j):
        wait(j, rows0, sem0)
        consume(j, rows0)
        start(j + 2, rows0, sem0)
        wait(j + 1, rows1, sem1)
        consume(j + 1, rows1)
        start(j + 3, rows1, sem1)

    wait(trip - 2, rows0, sem0)
    consume(trip - 2, rows0)
    wait(trip - 1, rows1, sem1)
    consume(trip - 1, rows1)

    plsc.subcore_barrier()

    for j in range(WB):
        pltpu.sync_copy(acc.at[pl.ds(rbase + j * CH, CH)], rows0)
        pltpu.sync_copy(rows0, osum.at[c].at[pl.ds(rbase + j * CH, CH)])
    if with_counts:
        @pl.when(c == count_core)
        def _():
            for j in range(WB):
                pltpu.sync_copy(cnt.at[pl.ds(rbase + j * CH, CH)], cbuf)
                pltpu.sync_copy(cbuf, ocnt.at[pl.ds(rbase + j * CH, CH)])
    plsc.subcore_barrier()


@functools.partial(
    pl.kernel,
    out_type=[
        jax.ShapeDtypeStruct((NC, R, DH2), jnp.float32),
        jax.ShapeDtypeStruct((R,), jnp.float32),
        jax.ShapeDtypeStruct((NC, R, DH2), jnp.float32),
        jax.ShapeDtypeStruct((R,), jnp.float32),
    ],
    mesh=_mesh,
    scratch_types=[
        pltpu.VMEM((CPT, CH), jnp.int32),
        pltpu.VMEM((CPT, CH), jnp.int32),
        pltpu.VMEM((CH, DH2), jnp.float32),
        pltpu.VMEM((CH, DH2), jnp.float32),
        pltpu.SemaphoreType.DMA,
        pltpu.SemaphoreType.DMA,
        pltpu.VMEM((CH,), jnp.float32),
        pltpu.VMEM((CH, DH2), jnp.float32),
        pltpu.VMEM((CH,), jnp.float32),
        pltpu.VMEM((CH,), jnp.float32),
        pltpu.VMEM_SHARED((R, DH2), jnp.float32),
        pltpu.VMEM_SHARED((R,), jnp.float32),
    ],
    compiler_params=_sc_params,
)
def _sc_seg_both(ha, hp, srcw, dstw, srcc, dstc, z2, z1, o1,
                 osw, ocw, osc, occ,
                 idx_s, idx_d, rows0, rows1, sem0, sem1,
                 ones_v, zbuf, zc, cbuf, acc, cnt):
    c = lax.axis_index("c")
    s = lax.axis_index("s")
    pltpu.sync_copy(z2, zbuf)
    pltpu.sync_copy(z1, zc)
    pltpu.sync_copy(o1, ones_v)
    _seg_pass(ha, srcw, dstw, osw, ocw, 0,
              idx_s, idx_d, rows0, rows1, sem0, sem1,
              ones_v, zbuf, zc, cbuf, acc, cnt, c, s)
    _seg_pass(hp, srcc, dstc, osc, occ, 1,
              idx_s, idx_d, rows0, rows1, sem0, sem1,
              ones_v, zbuf, zc, cbuf, acc, cnt, c, s)


@functools.partial(
    pl.kernel,
    out_type=[jax.ShapeDtypeStruct((NC, R, DH2), jnp.float32)],
    mesh=_mesh,
    scratch_types=[
        pltpu.VMEM((CPT, CH), jnp.int32),
        pltpu.VMEM((CPT, CH), jnp.int32),
        pltpu.VMEM((CH, DH2), jnp.float32),
        pltpu.VMEM((CH, DH2), jnp.float32),
        pltpu.SemaphoreType.DMA,
        pltpu.SemaphoreType.DMA,
        pltpu.VMEM((CH, DH2), jnp.float32),
        pltpu.VMEM_SHARED((R, DH2), jnp.float32),
    ],
    compiler_params=_sc_params,
)
def _sc_seg_cites(hp, srcc, dstc, z2, osc,
                  idx_s, idx_d, rows0, rows1, sem0, sem1, zbuf, acc):
    c = lax.axis_index("c")
    s = lax.axis_index("s")
    pltpu.sync_copy(z2, zbuf)
    _seg_pass(hp, srcc, dstc, osc, None, None,
              idx_s, idx_d, rows0, rows1, sem0, sem1,
              None, zbuf, None, None, acc, None, c, s)


def _proj_body(xa_ref, xp_ref, wa_ref, wp_ref, haf_ref, has_ref, hps_ref):
    ha = jnp.dot(xa_ref[...], wa_ref[...], preferred_element_type=jnp.float32)
    hp = jnp.dot(xp_ref[...], wp_ref[...], preferred_element_type=jnp.float32)
    haf_ref[...] = ha
    has_ref[0, ...] = ha[:, :DH2]
    has_ref[1, ...] = ha[:, DH2:]
    hps_ref[0, ...] = hp[:, :DH2]
    hps_ref[1, ...] = hp[:, DH2:]


def _project(xa, xp, wa, wp):
    return pl.pallas_call(
        _proj_body,
        grid=(GRID,),
        in_specs=[
            pl.BlockSpec((BM, D_IN), lambda i: (i, 0)),
            pl.BlockSpec((BM, D_IN), lambda i: (i, 0)),
            pl.BlockSpec((D_IN, D_H), lambda i: (0, 0)),
            pl.BlockSpec((D_IN, D_H), lambda i: (0, 0)),
        ],
        out_specs=[
            pl.BlockSpec((BM, D_H), lambda i: (i, 0)),
            pl.BlockSpec((NC, BM, DH2), lambda i: (0, i, 0)),
            pl.BlockSpec((NC, BM, DH2), lambda i: (0, i, 0)),
        ],
        out_shape=[
            jax.ShapeDtypeStruct((R, D_H), jnp.float32),
            jax.ShapeDtypeStruct((NC, R, DH2), jnp.float32),
            jax.ShapeDtypeStruct((NC, R, DH2), jnp.float32),
        ],
    )(xa, xp, wa, wp)


def _inv_counts_body(cw_ref, cc_ref, iw_ref, ic_ref):
    r0 = lax.broadcasted_iota(jnp.int32, (D_H, DH2), 0)
    r1 = lax.broadcasted_iota(jnp.int32, (D_H, DH2), 1)
    sel = (r0 == r1).astype(jnp.float32)
    for src, dst in ((cw_ref, iw_ref), (cc_ref, ic_ref)):
        inv8 = 1.0 / jnp.maximum(src[...], 1.0)
        for k in range(8):
            rb = jnp.broadcast_to(inv8[k:k + 1, :], (D_H, D_H))
            part = lax.dot_general(rb, sel, (((0,), (0,)), ((), ())),
                                   preferred_element_type=jnp.float32)
            dst[pl.ds(k * D_H, D_H), :] = part


def _inv_counts(cw, cc):
    return pl.pallas_call(
        _inv_counts_body,
        grid=(GRID,),
        in_specs=[
            pl.BlockSpec((8, D_H), lambda i: (i, 0)),
            pl.BlockSpec((8, D_H), lambda i: (i, 0)),
        ],
        out_specs=[
            pl.BlockSpec((BM, DH2), lambda i: (i, 0)),
            pl.BlockSpec((BM, DH2), lambda i: (i, 0)),
        ],
        out_shape=[
            jax.ShapeDtypeStruct((R, DH2), jnp.float32),
            jax.ShapeDtypeStruct((R, DH2), jnp.float32),
        ],
    )(cw.reshape(R // D_H, D_H), cc.reshape(R // D_H, D_H))


def _combine_body(sw_ref, iw_ref, sc_ref, ic_ref, hp_ref,
                  wlw_ref, wlc_ref, wrw_ref, wrc_ref, blw_ref, blc_ref,
                  out_ref, *, emit):
    iw = iw_ref[...]
    ic = ic_ref[...]
    wlw = wlw_ref[...]
    wlc = wlc_ref[...]
    wr = wrw_ref[...] + wrc_ref[...]
    dot = functools.partial(jnp.dot, preferred_element_type=jnp.float32)
    acc = dot(sw_ref[0] * iw, wlw[:DH2])
    acc = acc + dot(sw_ref[1] * iw, wlw[DH2:])
    acc = acc + dot(sc_ref[0] * ic, wlc[:DH2])
    acc = acc + dot(sc_ref[1] * ic, wlc[DH2:])
    acc = acc + dot(hp_ref[0], wr[:DH2])
    acc = acc + dot(hp_ref[1], wr[DH2:])
    acc = acc + blw_ref[...] + blc_ref[...]
    res = jnp.maximum(acc * 0.5, 0.0)
    if emit == "full":
        out_ref[...] = res
    else:
        out_ref[0, ...] = res[:, :DH2]
        out_ref[1, ...] = res[:, DH2:]


def _combine(sw, iw, sc, ic, hp, wlw, wlc, wrw, wrc, blw, blc, emit):
    if emit == "full":
        out_spec = pl.BlockSpec((BM, D_H), lambda i: (i, 0))
        out_shape = jax.ShapeDtypeStruct((R, D_H), jnp.float32)
    else:
        out_spec = pl.BlockSpec((NC, BM, DH2), lambda i: (0, i, 0))
        out_shape = jax.ShapeDtypeStruct((NC, R, DH2), jnp.float32)
    return pl.pallas_call(
        functools.partial(_combine_body, emit=emit),
        grid=(GRID,),
        in_specs=[
            pl.BlockSpec((NC, BM, DH2), lambda i: (0, i, 0)),
            pl.BlockSpec((BM, DH2), lambda i: (i, 0)),
            pl.BlockSpec((NC, BM, DH2), lambda i: (0, i, 0)),
            pl.BlockSpec((BM, DH2), lambda i: (i, 0)),
            pl.BlockSpec((NC, BM, DH2), lambda i: (0, i, 0)),
            pl.BlockSpec((D_H, D_H), lambda i: (0, 0)),
            pl.BlockSpec((D_H, D_H), lambda i: (0, 0)),
            pl.BlockSpec((D_H, D_H), lambda i: (0, 0)),
            pl.BlockSpec((D_H, D_H), lambda i: (0, 0)),
            pl.BlockSpec((1, D_H), lambda i: (0, 0)),
            pl.BlockSpec((1, D_H), lambda i: (0, 0)),
        ],
        out_specs=out_spec,
        out_shape=out_shape,
    )(sw, iw, sc, ic, hp, wlw, wlc, wrw, wrc, blw, blc)


def _prep_edges(ei):
    pad = EP - E
    src = jnp.concatenate([ei[0], jnp.zeros((pad,), jnp.int32)])
    dst = jnp.concatenate([ei[1], jnp.zeros((pad,), jnp.int32)])
    return src.reshape(NS, CPT, CH), dst.reshape(NS, CPT, CH)


def kernel(x_author, x_paper, edge_index_writes, edge_index_cites,
           W_in_author, W_in_paper,
           Wl_w0, bl_w0, Wr_w0, Wl_c0, bl_c0, Wr_c0,
           Wl_w1, bl_w1, Wr_w1, Wl_c1, bl_c1, Wr_c1):
    pad_rows = R - N_NODE
    xa = jnp.pad(x_author, ((0, pad_rows), (0, 0)))
    xp = jnp.pad(x_paper, ((0, pad_rows), (0, 0)))
    srcw, dstw = _prep_edges(edge_index_writes)
    srcc, dstc = _prep_edges(edge_index_cites)
    z2 = jnp.zeros((CH, DH2), jnp.float32)
    z1 = jnp.zeros((CH,), jnp.float32)
    o1 = jnp.ones((CH,), jnp.float32)

    haf, has, hp0s = _project(xa, xp, W_in_author, W_in_paper)

    sw, cw, sc0, cc = _sc_seg_both(has, hp0s, srcw, dstw, srcc, dstc,
                                   z2, z1, o1)
    iw, ic = _inv_counts(cw, cc)

    blw0 = bl_w0.reshape(1, D_H)
    blc0 = bl_c0.reshape(1, D_H)
    hp1s = _combine(sw, iw, sc0, ic, hp0s,
                    Wl_w0, Wl_c0, Wr_w0, Wr_c0, blw0, blc0, "split")

    (sc1,) = _sc_seg_cites(hp1s, srcc, dstc, z2)

    blw1 = bl_w1.reshape(1, D_H)
    blc1 = bl_c1.reshape(1, D_H)
    hp2 = _combine(sw, iw, sc1, ic, hp1s,
                   Wl_w1, Wl_c1, Wr_w1, Wr_c1, blw1, blc1, "full")

    return (haf[:N_NODE], hp2[:N_NODE])


# speedup vs baseline: 9.9134x; 1.0650x over previous
import functools

import jax
import jax.numpy as jnp
from jax import lax
from jax.experimental import pallas as pl
from jax.experimental.pallas import tpu as pltpu
from jax.experimental.pallas import tpu_sc as plsc

N_NODE = 10000
D_IN = 256
D_H = 128
DH2 = D_H // 2
E = 320000

NC = 2
NS = 16
CH = 128
CPT = 160
EP = NS * CPT * CH
NCH_REAL = E // CH
CPT_LAST = NCH_REAL - (NS - 1) * CPT
R = 10240
RPT = R // NS
WB = RPT // CH
BM = 1024
GRID = R // BM

_mesh = plsc.VectorSubcoreMesh(core_axis_name="c", subcore_axis_name="s")
_sc_params = pltpu.CompilerParams(use_tc_tiling_on_sc=False)


def _seg_pass(h_hbm, src_hbm, dst_hbm, osum, ocnt, count_core,
              idx_s, idx_d, bufs, sbufs,
              ones_v, zbuf, zc, cbuf, acc, cnt, c, s):
    rbase = s * RPT
    with_counts = ocnt is not None

    for j in range(WB):
        pltpu.sync_copy(zbuf, acc.at[pl.ds(rbase + j * CH, CH)])
    if with_counts:
        for j in range(WB):
            pltpu.sync_copy(zc, cnt.at[pl.ds(rbase + j * CH, CH)])
    plsc.subcore_barrier()

    pltpu.sync_copy(src_hbm.at[s], idx_s)
    pltpu.sync_copy(dst_hbm.at[s], idx_d)

    h_half = h_hbm.at[c]
    trip = jnp.where(s == NS - 1, CPT_LAST, CPT)
    nb = len(bufs)

    def start_g(j, rows, sem):
        pltpu.async_copy(h_half.at[idx_s.at[j]], rows, sem)

    def wait_g(j, rows, sem):
        pltpu.make_async_copy(h_half.at[idx_s.at[j]], rows, sem).wait()

    def start_s(j, rows, sem):
        pltpu.async_copy(rows, acc.at[idx_d.at[j]], sem, add=True)
        if with_counts:
            @pl.when(c == count_core)
            def _():
                pltpu.sync_copy(ones_v, cnt.at[idx_d.at[j]], add=True)

    def wait_s(j, rows, sem):
        pltpu.make_async_copy(rows, acc.at[idx_d.at[j]], sem).wait()

    for k in range(nb):
        start_g(k, *bufs[k])

    @pl.loop(0, trip - nb, step=nb)
    def _(j):
        for k in range(nb):
            rows, gsem, ssem = bufs[k][0], bufs[k][1], sbufs[k]
            wait_g(j + k, rows, gsem)
            start_s(j + k, rows, ssem)
        for k in range(nb):
            rows, gsem, ssem = bufs[k][0], bufs[k][1], sbufs[k]
            wait_s(j + k, rows, ssem)
            start_g(j + k + nb, rows, gsem)

    base = trip - nb
    for k in range(nb):
        rows, gsem, ssem = bufs[k][0], bufs[k][1], sbufs[k]
        wait_g(base + k, rows, gsem)
        start_s(base + k, rows, ssem)
    for k in range(nb):
        rows, gsem, ssem = bufs[k][0], bufs[k][1], sbufs[k]
        wait_s(base + k, rows, ssem)

    plsc.subcore_barrier()

    wrows = bufs[0][0]
    for j in range(WB):
        pltpu.sync_copy(acc.at[pl.ds(rbase + j * CH, CH)], wrows)
        pltpu.sync_copy(wrows, osum.at[c].at[pl.ds(rbase + j * CH, CH)])
    if with_counts:
        @pl.when(c == count_core)
        def _():
            for j in range(WB):
                pltpu.sync_copy(cnt.at[pl.ds(rbase + j * CH, CH)], cbuf)
                pltpu.sync_copy(cbuf, ocnt.at[pl.ds(rbase + j * CH, CH)])
    plsc.subcore_barrier()


@functools.partial(
    pl.kernel,
    out_type=[
        jax.ShapeDtypeStruct((NC, R, DH2), jnp.float32),
        jax.ShapeDtypeStruct((R,), jnp.float32),
        jax.ShapeDtypeStruct((NC, R, DH2), jnp.float32),
        jax.ShapeDtypeStruct((R,), jnp.float32),
    ],
    mesh=_mesh,
    scratch_types=[
        pltpu.VMEM((CPT, CH), jnp.int32),
        pltpu.VMEM((CPT, CH), jnp.int32),
        pltpu.VMEM((CH, DH2), jnp.float32),
        pltpu.VMEM((CH, DH2), jnp.float32),
        pltpu.VMEM((CH, DH2), jnp.float32),
        pltpu.VMEM((CH, DH2), jnp.float32),
        pltpu.SemaphoreType.DMA,
        pltpu.SemaphoreType.DMA,
        pltpu.SemaphoreType.DMA,
        pltpu.SemaphoreType.DMA,
        pltpu.SemaphoreType.DMA,
        pltpu.SemaphoreType.DMA,
        pltpu.SemaphoreType.DMA,
        pltpu.SemaphoreType.DMA,
        pltpu.VMEM((CH,), jnp.float32),
        pltpu.VMEM((CH, DH2), jnp.float32),
        pltpu.VMEM((CH,), jnp.float32),
        pltpu.VMEM((CH,), jnp.float32),
        pltpu.VMEM_SHARED((R, DH2), jnp.float32),
        pltpu.VMEM_SHARED((R,), jnp.float32),
    ],
    compiler_params=_sc_params,
)
def _sc_seg_both(ha, hp, srcw, dstw, srcc, dstc, z2, z1, o1,
                 osw, ocw, osc, occ,
                 idx_s, idx_d, r0, r1, r2, r3, g0, g1, g2, g3,
                 s0, s1, s2, s3,
                 ones_v, zbuf, zc, cbuf, acc, cnt):
    c = lax.axis_index("c")
    s = lax.axis_index("s")
    bufs = [(r0, g0), (r1, g1), (r2, g2), (r3, g3)]
    sbufs = [s0, s1, s2, s3]
    pltpu.sync_copy(z2, zbuf)
    pltpu.sync_copy(z1, zc)
    pltpu.sync_copy(o1, ones_v)
    _seg_pass(ha, srcw, dstw, osw, ocw, 0,
              idx_s, idx_d, bufs, sbufs,
              ones_v, zbuf, zc, cbuf, acc, cnt, c, s)
    _seg_pass(hp, srcc, dstc, osc, occ, 1,
              idx_s, idx_d, bufs, sbufs,
              ones_v, zbuf, zc, cbuf, acc, cnt, c, s)


@functools.partial(
    pl.kernel,
    out_type=[jax.ShapeDtypeStruct((NC, R, DH2), jnp.float32)],
    mesh=_mesh,
    scratch_types=[
        pltpu.VMEM((CPT, CH), jnp.int32),
        pltpu.VMEM((CPT, CH), jnp.int32),
        pltpu.VMEM((CH, DH2), jnp.float32),
        pltpu.VMEM((CH, DH2), jnp.float32),
        pltpu.VMEM((CH, DH2), jnp.float32),
        pltpu.VMEM((CH, DH2), jnp.float32),
        pltpu.SemaphoreType.DMA,
        pltpu.SemaphoreType.DMA,
        pltpu.SemaphoreType.DMA,
        pltpu.SemaphoreType.DMA,
        pltpu.SemaphoreType.DMA,
        pltpu.SemaphoreType.DMA,
        pltpu.SemaphoreType.DMA,
        pltpu.SemaphoreType.DMA,
        pltpu.VMEM((CH, DH2), jnp.float32),
        pltpu.VMEM_SHARED((R, DH2), jnp.float32),
    ],
    compiler_params=_sc_params,
)
def _sc_seg_cites(hp, srcc, dstc, z2, osc,
                  idx_s, idx_d, r0, r1, r2, r3, g0, g1, g2, g3,
                  s0, s1, s2, s3, zbuf, acc):
    c = lax.axis_index("c")
    s = lax.axis_index("s")
    bufs = [(r0, g0), (r1, g1), (r2, g2), (r3, g3)]
    sbufs = [s0, s1, s2, s3]
    pltpu.sync_copy(z2, zbuf)
    _seg_pass(hp, srcc, dstc, osc, None, None,
              idx_s, idx_d, bufs, sbufs,
              None, zbuf, None, None, acc, None, c, s)


def _proj_body(xa_ref, xp_ref, wa_ref, wp_ref, haf_ref, has_ref, hps_ref):
    ha = jnp.dot(xa_ref[...], wa_ref[...], preferred_element_type=jnp.float32)
    hp = jnp.dot(xp_ref[...], wp_ref[...], preferred_element_type=jnp.float32)
    haf_ref[...] = ha
    has_ref[0, ...] = ha[:, :DH2]
    has_ref[1, ...] = ha[:, DH2:]
    hps_ref[0, ...] = hp[:, :DH2]
    hps_ref[1, ...] = hp[:, DH2:]


def _project(xa, xp, wa, wp):
    return pl.pallas_call(
        _proj_body,
        grid=(GRID,),
        in_specs=[
            pl.BlockSpec((BM, D_IN), lambda i: (i, 0)),
            pl.BlockSpec((BM, D_IN), lambda i: (i, 0)),
            pl.BlockSpec((D_IN, D_H), lambda i: (0, 0)),
            pl.BlockSpec((D_IN, D_H), lambda i: (0, 0)),
        ],
        out_specs=[
            pl.BlockSpec((BM, D_H), lambda i: (i, 0)),
            pl.BlockSpec((NC, BM, DH2), lambda i: (0, i, 0)),
            pl.BlockSpec((NC, BM, DH2), lambda i: (0, i, 0)),
        ],
        out_shape=[
            jax.ShapeDtypeStruct((N_NODE, D_H), jnp.float32),
            jax.ShapeDtypeStruct((NC, N_NODE, DH2), jnp.float32),
            jax.ShapeDtypeStruct((NC, N_NODE, DH2), jnp.float32),
        ],
    )(xa, xp, wa, wp)


def _inv_counts_body(cw_ref, cc_ref, iw_ref, ic_ref):
    r0 = lax.broadcasted_iota(jnp.int32, (D_H, DH2), 0)
    r1 = lax.broadcasted_iota(jnp.int32, (D_H, DH2), 1)
    sel = (r0 == r1).astype(jnp.float32)
    for src, dst in ((cw_ref, iw_ref), (cc_ref, ic_ref)):
        inv8 = 1.0 / jnp.maximum(src[...], 1.0)
        for k in range(8):
            rb = jnp.broadcast_to(inv8[k:k + 1, :], (D_H, D_H))
            part = lax.dot_general(rb, sel, (((0,), (0,)), ((), ())),
                                   preferred_element_type=jnp.float32)
            dst[pl.ds(k * D_H, D_H), :] = part


def _inv_counts(cw, cc):
    return pl.pallas_call(
        _inv_counts_body,
        grid=(GRID,),
        in_specs=[
            pl.BlockSpec((8, D_H), lambda i: (i, 0)),
            pl.BlockSpec((8, D_H), lambda i: (i, 0)),
        ],
        out_specs=[
            pl.BlockSpec((BM, DH2), lambda i: (i, 0)),
            pl.BlockSpec((BM, DH2), lambda i: (i, 0)),
        ],
        out_shape=[
            jax.ShapeDtypeStruct((R, DH2), jnp.float32),
            jax.ShapeDtypeStruct((R, DH2), jnp.float32),
        ],
    )(cw.reshape(R // D_H, D_H), cc.reshape(R // D_H, D_H))


def _combine_body(sw_ref, iw_ref, sc_ref, ic_ref, hp_ref,
                  wlw_ref, wlc_ref, wrw_ref, wrc_ref, blw_ref, blc_ref,
                  out_ref, *, emit):
    iw = iw_ref[...]
    ic = ic_ref[...]
    wlw = wlw_ref[...]
    wlc = wlc_ref[...]
    wr = wrw_ref[...] + wrc_ref[...]
    dot = functools.partial(jnp.dot, preferred_element_type=jnp.float32)
    acc = dot(sw_ref[0] * iw, wlw[:DH2])
    acc = acc + dot(sw_ref[1] * iw, wlw[DH2:])
    acc = acc + dot(sc_ref[0] * ic, wlc[:DH2])
    acc = acc + dot(sc_ref[1] * ic, wlc[DH2:])
    acc = acc + dot(hp_ref[0], wr[:DH2])
    acc = acc + dot(hp_ref[1], wr[DH2:])
    acc = acc + blw_ref[...] + blc_ref[...]
    res = jnp.maximum(acc * 0.5, 0.0)
    if emit == "full":
        out_ref[...] = res
    else:
        out_ref[0, ...] = res[:, :DH2]
        out_ref[1, ...] = res[:, DH2:]


def _combine(sw, iw, sc, ic, hp, wlw, wlc, wrw, wrc, blw, blc, emit):
    if emit == "full":
        out_spec = pl.BlockSpec((BM, D_H), lambda i: (i, 0))
        out_shape = jax.ShapeDtypeStruct((N_NODE, D_H), jnp.float32)
    else:
        out_spec = pl.BlockSpec((NC, BM, DH2), lambda i: (0, i, 0))
        out_shape = jax.ShapeDtypeStruct((NC, N_NODE, DH2), jnp.float32)
    return pl.pallas_call(
        functools.partial(_combine_body, emit=emit),
        grid=(GRID,),
        in_specs=[
            pl.BlockSpec((NC, BM, DH2), lambda i: (0, i, 0)),
            pl.BlockSpec((BM, DH2), lambda i: (i, 0)),
            pl.BlockSpec((NC, BM, DH2), lambda i: (0, i, 0)),
            pl.BlockSpec((BM, DH2), lambda i: (i, 0)),
            pl.BlockSpec((NC, BM, DH2), lambda i: (0, i, 0)),
            pl.BlockSpec((D_H, D_H), lambda i: (0, 0)),
            pl.BlockSpec((D_H, D_H), lambda i: (0, 0)),
            pl.BlockSpec((D_H, D_H), lambda i: (0, 0)),
            pl.BlockSpec((D_H, D_H), lambda i: (0, 0)),
            pl.BlockSpec((1, D_H), lambda i: (0, 0)),
            pl.BlockSpec((1, D_H), lambda i: (0, 0)),
        ],
        out_specs=out_spec,
        out_shape=out_shape,
    )(sw, iw, sc, ic, hp, wlw, wlc, wrw, wrc, blw, blc)


def _prep_edges(ei):
    pad = EP - E
    src = jnp.concatenate([ei[0], jnp.zeros((pad,), jnp.int32)])
    dst = jnp.concatenate([ei[1], jnp.zeros((pad,), jnp.int32)])
    return src.reshape(NS, CPT, CH), dst.reshape(NS, CPT, CH)


def kernel(x_author, x_paper, edge_index_writes, edge_index_cites,
           W_in_author, W_in_paper,
           Wl_w0, bl_w0, Wr_w0, Wl_c0, bl_c0, Wr_c0,
           Wl_w1, bl_w1, Wr_w1, Wl_c1, bl_c1, Wr_c1):
    srcw, dstw = _prep_edges(edge_index_writes)
    srcc, dstc = _prep_edges(edge_index_cites)
    z2 = jnp.zeros((CH, DH2), jnp.float32)
    z1 = jnp.zeros((CH,), jnp.float32)
    o1 = jnp.ones((CH,), jnp.float32)

    haf, has, hp0s = _project(x_author, x_paper, W_in_author, W_in_paper)

    sw, cw, sc0, cc = _sc_seg_both(has, hp0s, srcw, dstw, srcc, dstc,
                                   z2, z1, o1)
    iw, ic = _inv_counts(cw, cc)

    blw0 = bl_w0.reshape(1, D_H)
    blc0 = bl_c0.reshape(1, D_H)
    hp1s = _combine(sw, iw, sc0, ic, hp0s,
                    Wl_w0, Wl_c0, Wr_w0, Wr_c0, blw0, blc0, "split")

    (sc1,) = _sc_seg_cites(hp1s, srcc, dstc, z2)

    blw1 = bl_w1.reshape(1, D_H)
    blc1 = bl_c1.reshape(1, D_H)
    hp2 = _combine(sw, iw, sc1, ic, hp1s,
                   Wl_w1, Wl_c1, Wr_w1, Wr_c1, blw1, blc1, "full")

    return (haf, hp2)


# speedup vs baseline: 10.5751x; 1.0667x over previous
import functools

import jax
import jax.numpy as jnp
from jax import lax
from jax.experimental import pallas as pl
from jax.experimental.pallas import tpu as pltpu
from jax.experimental.pallas import tpu_sc as plsc

N_NODE = 10000
D_IN = 256
D_H = 128
DH2 = D_H // 2
E = 320000

NC = 2
NS = 16
CH = 128
CPT = 160
EP = NS * CPT * CH
NCH_REAL = E // CH
CPT_LAST = NCH_REAL - (NS - 1) * CPT
R = 10240
RPT = R // NS
WB = RPT // CH
BM = 1024
GRID = R // BM

_mesh = plsc.VectorSubcoreMesh(core_axis_name="c", subcore_axis_name="s")
_sc_params = pltpu.CompilerParams(use_tc_tiling_on_sc=False)


def _seg_pass(h_hbm, src_hbm, dst_hbm, osum, ocnt, count_core,
              idx_s, idx_d, bufs, sbufs, cbufs,
              ones_v, zbuf, zc, cbuf, acc, cnt, c, s):
    rbase = s * RPT
    with_counts = ocnt is not None

    for j in range(WB):
        pltpu.sync_copy(zbuf, acc.at[pl.ds(rbase + j * CH, CH)])
    if with_counts:
        for j in range(WB):
            pltpu.sync_copy(zc, cnt.at[pl.ds(rbase + j * CH, CH)])
    plsc.subcore_barrier()

    pltpu.sync_copy(src_hbm.at[s], idx_s)
    pltpu.sync_copy(dst_hbm.at[s], idx_d)

    h_half = h_hbm.at[c]
    trip = jnp.where(s == NS - 1, CPT_LAST, CPT)
    nb = len(bufs)

    def start_g(j, rows, sem):
        pltpu.async_copy(h_half.at[idx_s.at[j]], rows, sem)

    def wait_g(j, rows, sem):
        pltpu.make_async_copy(h_half.at[idx_s.at[j]], rows, sem).wait()

    def start_s(j, rows, sem, csem):
        pltpu.async_copy(rows, acc.at[idx_d.at[j]], sem, add=True)
        if with_counts:
            @pl.when(c == count_core)
            def _():
                pltpu.async_copy(ones_v, cnt.at[idx_d.at[j]], csem, add=True)

    def wait_s(j, rows, sem, csem):
        pltpu.make_async_copy(rows, acc.at[idx_d.at[j]], sem).wait()
        if with_counts:
            @pl.when(c == count_core)
            def _():
                pltpu.make_async_copy(ones_v, cnt.at[idx_d.at[j]], csem).wait()

    for k in range(nb):
        start_g(k, *bufs[k])

    @pl.loop(0, trip - nb, step=nb)
    def _(j):
        for k in range(nb):
            rows, gsem, ssem, csem = bufs[k][0], bufs[k][1], sbufs[k], cbufs[k]
            wait_g(j + k, rows, gsem)
            start_s(j + k, rows, ssem, csem)
        for k in range(nb):
            rows, gsem, ssem, csem = bufs[k][0], bufs[k][1], sbufs[k], cbufs[k]
            wait_s(j + k, rows, ssem, csem)
            start_g(j + k + nb, rows, gsem)

    base = trip - nb
    for k in range(nb):
        rows, gsem, ssem, csem = bufs[k][0], bufs[k][1], sbufs[k], cbufs[k]
        wait_g(base + k, rows, gsem)
        start_s(base + k, rows, ssem, csem)
    for k in range(nb):
        rows, gsem, ssem, csem = bufs[k][0], bufs[k][1], sbufs[k], cbufs[k]
        wait_s(base + k, rows, ssem, csem)

    plsc.subcore_barrier()

    wrows = bufs[0][0]
    for j in range(WB):
        pltpu.sync_copy(acc.at[pl.ds(rbase + j * CH, CH)], wrows)
        pltpu.sync_copy(wrows, osum.at[c].at[pl.ds(rbase + j * CH, CH)])
    if with_counts:
        @pl.when(c == count_core)
        def _():
            for j in range(WB):
                pltpu.sync_copy(cnt.at[pl.ds(rbase + j * CH, CH)], cbuf)
                pltpu.sync_copy(cbuf, ocnt.at[pl.ds(rbase + j * CH, CH)])
    plsc.subcore_barrier()


@functools.partial(
    pl.kernel,
    out_type=[
        jax.ShapeDtypeStruct((NC, R, DH2), jnp.float32),
        jax.ShapeDtypeStruct((R,), jnp.float32),
        jax.ShapeDtypeStruct((NC, R, DH2), jnp.float32),
        jax.ShapeDtypeStruct((R,), jnp.float32),
    ],
    mesh=_mesh,
    scratch_types=[
        pltpu.VMEM((CPT, CH), jnp.int32),
        pltpu.VMEM((CPT, CH), jnp.int32),
        pltpu.VMEM((CH, DH2), jnp.float32),
        pltpu.VMEM((CH, DH2), jnp.float32),
        pltpu.VMEM((CH, DH2), jnp.float32),
        pltpu.VMEM((CH, DH2), jnp.float32),
        pltpu.SemaphoreType.DMA,
        pltpu.SemaphoreType.DMA,
        pltpu.SemaphoreType.DMA,
        pltpu.SemaphoreType.DMA,
        pltpu.SemaphoreType.DMA,
        pltpu.SemaphoreType.DMA,
        pltpu.SemaphoreType.DMA,
        pltpu.SemaphoreType.DMA,
        pltpu.SemaphoreType.DMA,
        pltpu.SemaphoreType.DMA,
        pltpu.SemaphoreType.DMA,
        pltpu.SemaphoreType.DMA,
        pltpu.VMEM((CH,), jnp.float32),
        pltpu.VMEM((CH, DH2), jnp.float32),
        pltpu.VMEM((CH,), jnp.float32),
        pltpu.VMEM((CH,), jnp.float32),
        pltpu.VMEM_SHARED((R, DH2), jnp.float32),
        pltpu.VMEM_SHARED((R,), jnp.float32),
    ],
    compiler_params=_sc_params,
)
def _sc_seg_both(ha, hp, srcw, dstw, srcc, dstc, z2, z1, o1,
                 osw, ocw, osc, occ,
                 idx_s, idx_d, r0, r1, r2, r3, g0, g1, g2, g3,
                 s0, s1, s2, s3, c0, c1, c2, c3,
                 ones_v, zbuf, zc, cbuf, acc, cnt):
    c = lax.axis_index("c")
    s = lax.axis_index("s")
    bufs = [(r0, g0), (r1, g1), (r2, g2), (r3, g3)]
    sbufs = [s0, s1, s2, s3]
    cbufs = [c0, c1, c2, c3]
    pltpu.sync_copy(z2, zbuf)
    pltpu.sync_copy(z1, zc)
    pltpu.sync_copy(o1, ones_v)
    _seg_pass(ha, srcw, dstw, osw, ocw, 0,
              idx_s, idx_d, bufs, sbufs, cbufs,
              ones_v, zbuf, zc, cbuf, acc, cnt, c, s)
    _seg_pass(hp, srcc, dstc, osc, occ, 1,
              idx_s, idx_d, bufs, sbufs, cbufs,
              ones_v, zbuf, zc, cbuf, acc, cnt, c, s)


@functools.partial(
    pl.kernel,
    out_type=[jax.ShapeDtypeStruct((NC, R, DH2), jnp.float32)],
    mesh=_mesh,
    scratch_types=[
        pltpu.VMEM((CPT, CH), jnp.int32),
        pltpu.VMEM((CPT, CH), jnp.int32),
        pltpu.VMEM((CH, DH2), jnp.float32),
        pltpu.VMEM((CH, DH2), jnp.float32),
        pltpu.VMEM((CH, DH2), jnp.float32),
        pltpu.VMEM((CH, DH2), jnp.float32),
        pltpu.SemaphoreType.DMA,
        pltpu.SemaphoreType.DMA,
        pltpu.SemaphoreType.DMA,
        pltpu.SemaphoreType.DMA,
        pltpu.SemaphoreType.DMA,
        pltpu.SemaphoreType.DMA,
        pltpu.SemaphoreType.DMA,
        pltpu.SemaphoreType.DMA,
        pltpu.VMEM((CH, DH2), jnp.float32),
        pltpu.VMEM_SHARED((R, DH2), jnp.float32),
    ],
    compiler_params=_sc_params,
)
def _sc_seg_cites(hp, srcc, dstc, z2, osc,
                  idx_s, idx_d, r0, r1, r2, r3, g0, g1, g2, g3,
                  s0, s1, s2, s3, zbuf, acc):
    c = lax.axis_index("c")
    s = lax.axis_index("s")
    bufs = [(r0, g0), (r1, g1), (r2, g2), (r3, g3)]
    sbufs = [s0, s1, s2, s3]
    pltpu.sync_copy(z2, zbuf)
    _seg_pass(hp, srcc, dstc, osc, None, None,
              idx_s, idx_d, bufs, sbufs, [None] * 4,
              None, zbuf, None, None, acc, None, c, s)


def _proj_body(xa_ref, xp_ref, wa_ref, wp_ref, haf_ref, has_ref, hps_ref):
    ha = jnp.dot(xa_ref[...], wa_ref[...], preferred_element_type=jnp.float32)
    hp = jnp.dot(xp_ref[...], wp_ref[...], preferred_element_type=jnp.float32)
    haf_ref[...] = ha
    has_ref[0, ...] = ha[:, :DH2]
    has_ref[1, ...] = ha[:, DH2:]
    hps_ref[0, ...] = hp[:, :DH2]
    hps_ref[1, ...] = hp[:, DH2:]


def _project(xa, xp, wa, wp):
    return pl.pallas_call(
        _proj_body,
        grid=(GRID,),
        in_specs=[
            pl.BlockSpec((BM, D_IN), lambda i: (i, 0)),
            pl.BlockSpec((BM, D_IN), lambda i: (i, 0)),
            pl.BlockSpec((D_IN, D_H), lambda i: (0, 0)),
            pl.BlockSpec((D_IN, D_H), lambda i: (0, 0)),
        ],
        out_specs=[
            pl.BlockSpec((BM, D_H), lambda i: (i, 0)),
            pl.BlockSpec((NC, BM, DH2), lambda i: (0, i, 0)),
            pl.BlockSpec((NC, BM, DH2), lambda i: (0, i, 0)),
        ],
        out_shape=[
            jax.ShapeDtypeStruct((N_NODE, D_H), jnp.float32),
            jax.ShapeDtypeStruct((NC, N_NODE, DH2), jnp.float32),
            jax.ShapeDtypeStruct((NC, N_NODE, DH2), jnp.float32),
        ],
    )(xa, xp, wa, wp)


def _inv_counts_body(cw_ref, cc_ref, iw_ref, ic_ref):
    r0 = lax.broadcasted_iota(jnp.int32, (D_H, DH2), 0)
    r1 = lax.broadcasted_iota(jnp.int32, (D_H, DH2), 1)
    sel = (r0 == r1).astype(jnp.float32)
    for src, dst in ((cw_ref, iw_ref), (cc_ref, ic_ref)):
        inv8 = 1.0 / jnp.maximum(src[...], 1.0)
        for k in range(8):
            rb = jnp.broadcast_to(inv8[k:k + 1, :], (D_H, D_H))
            part = lax.dot_general(rb, sel, (((0,), (0,)), ((), ())),
                                   preferred_element_type=jnp.float32)
            dst[pl.ds(k * D_H, D_H), :] = part


def _inv_counts(cw, cc):
    return pl.pallas_call(
        _inv_counts_body,
        grid=(GRID,),
        in_specs=[
            pl.BlockSpec((8, D_H), lambda i: (i, 0)),
            pl.BlockSpec((8, D_H), lambda i: (i, 0)),
        ],
        out_specs=[
            pl.BlockSpec((BM, DH2), lambda i: (i, 0)),
            pl.BlockSpec((BM, DH2), lambda i: (i, 0)),
        ],
        out_shape=[
            jax.ShapeDtypeStruct((R, DH2), jnp.float32),
            jax.ShapeDtypeStruct((R, DH2), jnp.float32),
        ],
    )(cw.reshape(R // D_H, D_H), cc.reshape(R // D_H, D_H))


def _combine_body(sw_ref, iw_ref, sc_ref, ic_ref, hp_ref,
                  wlw_ref, wlc_ref, wrw_ref, wrc_ref, blw_ref, blc_ref,
                  out_ref, *, emit):
    iw = iw_ref[...]
    ic = ic_ref[...]
    wlw = wlw_ref[...]
    wlc = wlc_ref[...]
    wr = wrw_ref[...] + wrc_ref[...]
    dot = functools.partial(jnp.dot, preferred_element_type=jnp.float32)
    acc = dot(sw_ref[0] * iw, wlw[:DH2])
    acc = acc + dot(sw_ref[1] * iw, wlw[DH2:])
    acc = acc + dot(sc_ref[0] * ic, wlc[:DH2])
    acc = acc + dot(sc_ref[1] * ic, wlc[DH2:])
    acc = acc + dot(hp_ref[0], wr[:DH2])
    acc = acc + dot(hp_ref[1], wr[DH2:])
    acc = acc + blw_ref[...] + blc_ref[...]
    res = jnp.maximum(acc * 0.5, 0.0)
    if emit == "full":
        out_ref[...] = res
    else:
        out_ref[0, ...] = res[:, :DH2]
        out_ref[1, ...] = res[:, DH2:]


def _combine(sw, iw, sc, ic, hp, wlw, wlc, wrw, wrc, blw, blc, emit):
    if emit == "full":
        out_spec = pl.BlockSpec((BM, D_H), lambda i: (i, 0))
        out_shape = jax.ShapeDtypeStruct((N_NODE, D_H), jnp.float32)
    else:
        out_spec = pl.BlockSpec((NC, BM, DH2), lambda i: (0, i, 0))
        out_shape = jax.ShapeDtypeStruct((NC, N_NODE, DH2), jnp.float32)
    return pl.pallas_call(
        functools.partial(_combine_body, emit=emit),
        grid=(GRID,),
        in_specs=[
            pl.BlockSpec((NC, BM, DH2), lambda i: (0, i, 0)),
            pl.BlockSpec((BM, DH2), lambda i: (i, 0)),
            pl.BlockSpec((NC, BM, DH2), lambda i: (0, i, 0)),
            pl.BlockSpec((BM, DH2), lambda i: (i, 0)),
            pl.BlockSpec((NC, BM, DH2), lambda i: (0, i, 0)),
            pl.BlockSpec((D_H, D_H), lambda i: (0, 0)),
            pl.BlockSpec((D_H, D_H), lambda i: (0, 0)),
            pl.BlockSpec((D_H, D_H), lambda i: (0, 0)),
            pl.BlockSpec((D_H, D_H), lambda i: (0, 0)),
            pl.BlockSpec((1, D_H), lambda i: (0, 0)),
            pl.BlockSpec((1, D_H), lambda i: (0, 0)),
        ],
        out_specs=out_spec,
        out_shape=out_shape,
    )(sw, iw, sc, ic, hp, wlw, wlc, wrw, wrc, blw, blc)


def _prep_edges(ei):
    pad = EP - E
    src = jnp.concatenate([ei[0], jnp.zeros((pad,), jnp.int32)])
    dst = jnp.concatenate([ei[1], jnp.zeros((pad,), jnp.int32)])
    return src.reshape(NS, CPT, CH), dst.reshape(NS, CPT, CH)


def kernel(x_author, x_paper, edge_index_writes, edge_index_cites,
           W_in_author, W_in_paper,
           Wl_w0, bl_w0, Wr_w0, Wl_c0, bl_c0, Wr_c0,
           Wl_w1, bl_w1, Wr_w1, Wl_c1, bl_c1, Wr_c1):
    srcw, dstw = _prep_edges(edge_index_writes)
    srcc, dstc = _prep_edges(edge_index_cites)
    z2 = jnp.zeros((CH, DH2), jnp.float32)
    z1 = jnp.zeros((CH,), jnp.float32)
    o1 = jnp.ones((CH,), jnp.float32)

    haf, has, hp0s = _project(x_author, x_paper, W_in_author, W_in_paper)

    sw, cw, sc0, cc = _sc_seg_both(has, hp0s, srcw, dstw, srcc, dstc,
                                   z2, z1, o1)
    iw, ic = _inv_counts(cw, cc)

    blw0 = bl_w0.reshape(1, D_H)
    blc0 = bl_c0.reshape(1, D_H)
    hp1s = _combine(sw, iw, sc0, ic, hp0s,
                    Wl_w0, Wl_c0, Wr_w0, Wr_c0, blw0, blc0, "split")

    (sc1,) = _sc_seg_cites(hp1s, srcc, dstc, z2)

    blw1 = bl_w1.reshape(1, D_H)
    blc1 = bl_c1.reshape(1, D_H)
    hp2 = _combine(sw, iw, sc1, ic, hp1s,
                   Wl_w1, Wl_c1, Wr_w1, Wr_c1, blw1, blc1, "full")

    return (haf, hp2)


# speedup vs baseline: 11.2230x; 1.0613x over previous
import functools

import jax
import jax.numpy as jnp
from jax import lax
from jax.experimental import pallas as pl
from jax.experimental.pallas import tpu as pltpu
from jax.experimental.pallas import tpu_sc as plsc

N_NODE = 10000
D_IN = 256
D_H = 128
DH2 = D_H // 2
E = 320000

NC = 2
NS = 16
CH = 128
CPT = 160
EP = NS * CPT * CH
NCH_REAL = E // CH
CPT_LAST = NCH_REAL - (NS - 1) * CPT
R = 10240
RPT = R // NS
WB = RPT // CH
NB = 4
BM = 1024
GRID = R // BM

_mesh = plsc.VectorSubcoreMesh(core_axis_name="c", subcore_axis_name="s")
_sc_params = pltpu.CompilerParams(use_tc_tiling_on_sc=False)


def _seg_pass(h_hbm, src_hbm, dst_hbm, osum, ocnt, count_core,
              idx_s, idx_d, bufs, sbufs, cbufs,
              ones_v, zbuf, zc, cbuf, acc, cnt, c, s):
    rbase = s * RPT
    chalf = c * DH2
    with_counts = ocnt is not None

    for j in range(WB):
        pltpu.sync_copy(zbuf, acc.at[pl.ds(rbase + j * CH, CH)])
    if with_counts:
        for j in range(WB):
            pltpu.sync_copy(zc, cnt.at[pl.ds(rbase + j * CH, CH)])
    plsc.subcore_barrier()

    pltpu.sync_copy(src_hbm.at[s], idx_s)
    pltpu.sync_copy(dst_hbm.at[s], idx_d)

    trip = jnp.where(s == NS - 1, CPT_LAST, CPT)

    h_half = h_hbm.at[c]

    def start_g(j, rows, sem):
        pltpu.async_copy(h_half.at[idx_s.at[j]], rows, sem)

    def wait_g(j, rows, sem):
        pltpu.make_async_copy(h_half.at[idx_s.at[j]], rows, sem).wait()

    def start_s(j, rows, sem, csem):
        pltpu.async_copy(rows, acc.at[idx_d.at[j]], sem, add=True)
        if with_counts:
            @pl.when(c == count_core)
            def _():
                pltpu.async_copy(ones_v, cnt.at[idx_d.at[j]], csem, add=True)

    def wait_s(j, rows, sem, csem):
        pltpu.make_async_copy(rows, acc.at[idx_d.at[j]], sem).wait()
        if with_counts:
            @pl.when(c == count_core)
            def _():
                pltpu.make_async_copy(ones_v, cnt.at[idx_d.at[j]], csem).wait()

    for k in range(NB):
        start_g(k, *bufs[k])

    @pl.loop(0, trip - NB, step=NB)
    def _(j):
        for k in range(NB):
            rows, gsem, ssem, csem = bufs[k][0], bufs[k][1], sbufs[k], cbufs[k]
            wait_g(j + k, rows, gsem)
            start_s(j + k, rows, ssem, csem)
        for k in range(NB):
            rows, gsem, ssem, csem = bufs[k][0], bufs[k][1], sbufs[k], cbufs[k]
            wait_s(j + k, rows, ssem, csem)
            start_g(j + k + NB, rows, gsem)

    base = trip - NB
    for k in range(NB):
        rows, gsem, ssem, csem = bufs[k][0], bufs[k][1], sbufs[k], cbufs[k]
        wait_g(base + k, rows, gsem)
        start_s(base + k, rows, ssem, csem)
    for k in range(NB):
        rows, gsem, ssem, csem = bufs[k][0], bufs[k][1], sbufs[k], cbufs[k]
        wait_s(base + k, rows, ssem, csem)

    plsc.subcore_barrier()

    wrows = bufs[0][0]
    for j in range(WB):
        pltpu.sync_copy(acc.at[pl.ds(rbase + j * CH, CH)], wrows)
        pltpu.sync_copy(
            wrows, osum.at[pl.ds(rbase + j * CH, CH), pl.ds(chalf, DH2)])
    if with_counts:
        @pl.when(c == count_core)
        def _():
            for j in range(WB):
                pltpu.sync_copy(cnt.at[pl.ds(rbase + j * CH, CH)], cbuf)
                pltpu.sync_copy(cbuf, ocnt.at[pl.ds(rbase + j * CH, CH)])
    plsc.subcore_barrier()


@functools.partial(
    pl.kernel,
    out_type=[
        jax.ShapeDtypeStruct((R, D_H), jnp.float32),
        jax.ShapeDtypeStruct((R,), jnp.float32),
        jax.ShapeDtypeStruct((R, D_H), jnp.float32),
        jax.ShapeDtypeStruct((R,), jnp.float32),
    ],
    mesh=_mesh,
    scratch_types=[
        pltpu.VMEM((CPT, CH), jnp.int32),
        pltpu.VMEM((CPT, CH), jnp.int32),
        pltpu.VMEM((CH, DH2), jnp.float32),
        pltpu.VMEM((CH, DH2), jnp.float32),
        pltpu.VMEM((CH, DH2), jnp.float32),
        pltpu.VMEM((CH, DH2), jnp.float32),
        pltpu.SemaphoreType.DMA,
        pltpu.SemaphoreType.DMA,
        pltpu.SemaphoreType.DMA,
        pltpu.SemaphoreType.DMA,
        pltpu.SemaphoreType.DMA,
        pltpu.SemaphoreType.DMA,
        pltpu.SemaphoreType.DMA,
        pltpu.SemaphoreType.DMA,
        pltpu.SemaphoreType.DMA,
        pltpu.SemaphoreType.DMA,
        pltpu.SemaphoreType.DMA,
        pltpu.SemaphoreType.DMA,
        pltpu.VMEM((CH,), jnp.float32),
        pltpu.VMEM((CH, DH2), jnp.float32),
        pltpu.VMEM((CH,), jnp.float32),
        pltpu.VMEM((CH,), jnp.float32),
        pltpu.VMEM_SHARED((R, DH2), jnp.float32),
        pltpu.VMEM_SHARED((R,), jnp.float32),
    ],
    compiler_params=_sc_params,
)
def _sc_seg_both(ha, hp, srcw, dstw, srcc, dstc, z2, z1, o1,
                 osw, ocw, osc, occ,
                 idx_s, idx_d, r0, r1, r2, r3, g0, g1, g2, g3,
                 s0, s1, s2, s3, c0, c1, c2, c3,
                 ones_v, zbuf, zc, cbuf, acc, cnt):
    c = lax.axis_index("c")
    s = lax.axis_index("s")
    bufs = [(r0, g0), (r1, g1), (r2, g2), (r3, g3)]
    sbufs = [s0, s1, s2, s3]
    cbufs = [c0, c1, c2, c3]
    pltpu.sync_copy(z2, zbuf)
    pltpu.sync_copy(z1, zc)
    pltpu.sync_copy(o1, ones_v)
    _seg_pass(ha, srcw, dstw, osw, ocw, 0,
              idx_s, idx_d, bufs, sbufs, cbufs,
              ones_v, zbuf, zc, cbuf, acc, cnt, c, s)
    _seg_pass(hp, srcc, dstc, osc, occ, 1,
              idx_s, idx_d, bufs, sbufs, cbufs,
              ones_v, zbuf, zc, cbuf, acc, cnt, c, s)


@functools.partial(
    pl.kernel,
    out_type=[jax.ShapeDtypeStruct((R, D_H), jnp.float32)],
    mesh=_mesh,
    scratch_types=[
        pltpu.VMEM((CPT, CH), jnp.int32),
        pltpu.VMEM((CPT, CH), jnp.int32),
        pltpu.VMEM((CH, DH2), jnp.float32),
        pltpu.VMEM((CH, DH2), jnp.float32),
        pltpu.VMEM((CH, DH2), jnp.float32),
        pltpu.VMEM((CH, DH2), jnp.float32),
        pltpu.SemaphoreType.DMA,
        pltpu.SemaphoreType.DMA,
        pltpu.SemaphoreType.DMA,
        pltpu.SemaphoreType.DMA,
        pltpu.SemaphoreType.DMA,
        pltpu.SemaphoreType.DMA,
        pltpu.SemaphoreType.DMA,
        pltpu.SemaphoreType.DMA,
        pltpu.VMEM((CH, DH2), jnp.float32),
        pltpu.VMEM_SHARED((R, DH2), jnp.float32),
    ],
    compiler_params=_sc_params,
)
def _sc_seg_cites(hp, srcc, dstc, z2, osc,
                  idx_s, idx_d, r0, r1, r2, r3, g0, g1, g2, g3,
                  s0, s1, s2, s3, zbuf, acc):
    c = lax.axis_index("c")
    s = lax.axis_index("s")
    bufs = [(r0, g0), (r1, g1), (r2, g2), (r3, g3)]
    sbufs = [s0, s1, s2, s3]
    pltpu.sync_copy(z2, zbuf)
    _seg_pass(hp, srcc, dstc, osc, None, None,
              idx_s, idx_d, bufs, sbufs, [None] * NB,
              None, zbuf, None, None, acc, None, c, s)


def _proj_body(xa_ref, xp_ref, wa_ref, wp_ref, ha_ref, has_ref, hps_ref):
    ha = jnp.dot(xa_ref[...], wa_ref[...], preferred_element_type=jnp.float32)
    hp = jnp.dot(xp_ref[...], wp_ref[...], preferred_element_type=jnp.float32)
    ha_ref[...] = ha
    has_ref[0, ...] = ha[:, :DH2]
    has_ref[1, ...] = ha[:, DH2:]
    hps_ref[0, ...] = hp[:, :DH2]
    hps_ref[1, ...] = hp[:, DH2:]


def _project(xa, xp, wa, wp):
    return pl.pallas_call(
        _proj_body,
        grid=(GRID,),
        in_specs=[
            pl.BlockSpec((BM, D_IN), lambda i: (i, 0)),
            pl.BlockSpec((BM, D_IN), lambda i: (i, 0)),
            pl.BlockSpec((D_IN, D_H), lambda i: (0, 0)),
            pl.BlockSpec((D_IN, D_H), lambda i: (0, 0)),
        ],
        out_specs=[
            pl.BlockSpec((BM, D_H), lambda i: (i, 0)),
            pl.BlockSpec((NC, BM, DH2), lambda i: (0, i, 0)),
            pl.BlockSpec((NC, BM, DH2), lambda i: (0, i, 0)),
        ],
        out_shape=[
            jax.ShapeDtypeStruct((N_NODE, D_H), jnp.float32),
            jax.ShapeDtypeStruct((NC, N_NODE, DH2), jnp.float32),
            jax.ShapeDtypeStruct((NC, N_NODE, DH2), jnp.float32),
        ],
    )(xa, xp, wa, wp)


def _inv_counts_body(cw_ref, cc_ref, iw_ref, ic_ref):
    r0 = lax.broadcasted_iota(jnp.int32, (D_H, D_H), 0)
    r1 = lax.broadcasted_iota(jnp.int32, (D_H, D_H), 1)
    sel = (r0 == r1).astype(jnp.float32)
    for src, dst in ((cw_ref, iw_ref), (cc_ref, ic_ref)):
        inv8 = 1.0 / jnp.maximum(src[...], 1.0)
        for k in range(8):
            rb = jnp.broadcast_to(inv8[k:k + 1, :], (D_H, D_H))
            part = lax.dot_general(rb, sel, (((0,), (0,)), ((), ())),
                                   preferred_element_type=jnp.float32)
            dst[pl.ds(k * D_H, D_H), :] = part


def _inv_counts(cw, cc):
    return pl.pallas_call(
        _inv_counts_body,
        grid=(GRID,),
        in_specs=[
            pl.BlockSpec((8, D_H), lambda i: (i, 0)),
            pl.BlockSpec((8, D_H), lambda i: (i, 0)),
        ],
        out_specs=[
            pl.BlockSpec((BM, D_H), lambda i: (i, 0)),
            pl.BlockSpec((BM, D_H), lambda i: (i, 0)),
        ],
        out_shape=[
            jax.ShapeDtypeStruct((R, D_H), jnp.float32),
            jax.ShapeDtypeStruct((R, D_H), jnp.float32),
        ],
    )(cw.reshape(R // D_H, D_H), cc.reshape(R // D_H, D_H))


def _combine_body(sw_ref, iw_ref, sc_ref, ic_ref, hp_ref,
                  wlw_ref, wlc_ref, wrw_ref, wrc_ref, blw_ref, blc_ref,
                  out_ref, *, emit):
    dot = functools.partial(jnp.dot, preferred_element_type=jnp.float32)
    wr = wrw_ref[...] + wrc_ref[...]
    acc = dot(sw_ref[...] * iw_ref[...], wlw_ref[...])
    acc = acc + dot(sc_ref[...] * ic_ref[...], wlc_ref[...])
    acc = acc + dot(hp_ref[0], wr[:DH2])
    acc = acc + dot(hp_ref[1], wr[DH2:])
    acc = acc + blw_ref[...] + blc_ref[...]
    res = jnp.maximum(acc * 0.5, 0.0)
    if emit == "full":
        out_ref[...] = res
    else:
        out_ref[0, ...] = res[:, :DH2]
        out_ref[1, ...] = res[:, DH2:]


def _combine(sw, iw, sc, ic, hp, wlw, wlc, wrw, wrc, blw, blc, emit):
    if emit == "full":
        out_spec = pl.BlockSpec((BM, D_H), lambda i: (i, 0))
        out_shape = jax.ShapeDtypeStruct((N_NODE, D_H), jnp.float32)
    else:
        out_spec = pl.BlockSpec((NC, BM, DH2), lambda i: (0, i, 0))
        out_shape = jax.ShapeDtypeStruct((NC, N_NODE, DH2), jnp.float32)
    return pl.pallas_call(
        functools.partial(_combine_body, emit=emit),
        grid=(GRID,),
        in_specs=[
            pl.BlockSpec((BM, D_H), lambda i: (i, 0)),
            pl.BlockSpec((BM, D_H), lambda i: (i, 0)),
            pl.BlockSpec((BM, D_H), lambda i: (i, 0)),
            pl.BlockSpec((BM, D_H), lambda i: (i, 0)),
            pl.BlockSpec((NC, BM, DH2), lambda i: (0, i, 0)),
            pl.BlockSpec((D_H, D_H), lambda i: (0, 0)),
            pl.BlockSpec((D_H, D_H), lambda i: (0, 0)),
            pl.BlockSpec((D_H, D_H), lambda i: (0, 0)),
            pl.BlockSpec((D_H, D_H), lambda i: (0, 0)),
            pl.BlockSpec((1, D_H), lambda i: (0, 0)),
            pl.BlockSpec((1, D_H), lambda i: (0, 0)),
        ],
        out_specs=out_spec,
        out_shape=out_shape,
    )(sw, iw, sc, ic, hp, wlw, wlc, wrw, wrc, blw, blc)


def _prep_edges(ei):
    pad = EP - E
    src = jnp.concatenate([ei[0], jnp.zeros((pad,), jnp.int32)])
    dst = jnp.concatenate([ei[1], jnp.zeros((pad,), jnp.int32)])
    return src.reshape(NS, CPT, CH), dst.reshape(NS, CPT, CH)


def kernel(x_author, x_paper, edge_index_writes, edge_index_cites,
           W_in_author, W_in_paper,
           Wl_w0, bl_w0, Wr_w0, Wl_c0, bl_c0, Wr_c0,
           Wl_w1, bl_w1, Wr_w1, Wl_c1, bl_c1, Wr_c1):
    srcw, dstw = _prep_edges(edge_index_writes)
    srcc, dstc = _prep_edges(edge_index_cites)
    z2 = jnp.zeros((CH, DH2), jnp.float32)
    z1 = jnp.zeros((CH,), jnp.float32)
    o1 = jnp.ones((CH,), jnp.float32)

    haf, has, hp0s = _project(x_author, x_paper, W_in_author, W_in_paper)

    sw, cw, sc0, cc = _sc_seg_both(has, hp0s, srcw, dstw, srcc, dstc,
                                   z2, z1, o1)
    iw, ic = _inv_counts(cw, cc)

    blw0 = bl_w0.reshape(1, D_H)
    blc0 = bl_c0.reshape(1, D_H)
    hp1s = _combine(sw, iw, sc0, ic, hp0s,
                    Wl_w0, Wl_c0, Wr_w0, Wr_c0, blw0, blc0, "split")

    (sc1,) = _sc_seg_cites(hp1s, srcc, dstc, z2)

    blw1 = bl_w1.reshape(1, D_H)
    blc1 = bl_c1.reshape(1, D_H)
    hp2 = _combine(sw, iw, sc1, ic, hp1s,
                   Wl_w1, Wl_c1, Wr_w1, Wr_c1, blw1, blc1, "full")

    return (haf, hp2)


# speedup vs baseline: 12.0266x; 1.0716x over previous
import functools

import jax
import jax.numpy as jnp
from jax import lax
from jax.experimental import pallas as pl
from jax.experimental.pallas import tpu as pltpu
from jax.experimental.pallas import tpu_sc as plsc

N_NODE = 10000
D_IN = 256
D_H = 128
DH2 = D_H // 2
E = 320000

NC = 2
NS = 16
CH = 128
CPT = 160
EP = NS * CPT * CH
NCH_REAL = E // CH
CPT_LAST = NCH_REAL - (NS - 1) * CPT
R = 10240
RPT = R // NS
WB = RPT // CH
NB = 4
BM = 1024
GRID = R // BM

_mesh = plsc.VectorSubcoreMesh(core_axis_name="c", subcore_axis_name="s")
_sc_params = pltpu.CompilerParams(use_tc_tiling_on_sc=False)


def _seg_pass(h_hbm, src_hbm, dst_hbm, osum, ocnt, count_core,
              idx_s, idx_d, bufs, sbufs, cbufs,
              ones_v, zbuf, zc, cbuf, acc, cnt, c, s):
    rbase = s * RPT
    chalf = c * DH2
    with_counts = ocnt is not None

    for j in range(WB):
        pltpu.sync_copy(zbuf, acc.at[pl.ds(rbase + j * CH, CH)])
    if with_counts:
        for j in range(WB):
            pltpu.sync_copy(zc, cnt.at[pl.ds(rbase + j * CH, CH)])
    plsc.subcore_barrier()

    pltpu.sync_copy(src_hbm.at[s], idx_s)
    pltpu.sync_copy(dst_hbm.at[s], idx_d)

    trip = jnp.where(s == NS - 1, CPT_LAST, CPT)

    h_half = h_hbm.at[pl.ds(c, 2 * N_NODE - 1)]

    def start_g(j, rows, sem):
        pltpu.async_copy(h_half.at[idx_s.at[j]], rows, sem)

    def wait_g(j, rows, sem):
        pltpu.make_async_copy(h_half.at[idx_s.at[j]], rows, sem).wait()

    def start_s(j, rows, sem, csem):
        pltpu.async_copy(rows, acc.at[idx_d.at[j]], sem, add=True)
        if with_counts:
            @pl.when(c == count_core)
            def _():
                pltpu.async_copy(ones_v, cnt.at[idx_d.at[j]], csem, add=True)

    def wait_s(j, rows, sem, csem):
        pltpu.make_async_copy(rows, acc.at[idx_d.at[j]], sem).wait()
        if with_counts:
            @pl.when(c == count_core)
            def _():
                pltpu.make_async_copy(ones_v, cnt.at[idx_d.at[j]], csem).wait()

    for k in range(NB):
        start_g(k, *bufs[k])

    @pl.loop(0, trip - NB, step=NB)
    def _(j):
        for k in range(NB):
            rows, gsem, ssem, csem = bufs[k][0], bufs[k][1], sbufs[k], cbufs[k]
            wait_g(j + k, rows, gsem)
            start_s(j + k, rows, ssem, csem)
        for k in range(NB):
            rows, gsem, ssem, csem = bufs[k][0], bufs[k][1], sbufs[k], cbufs[k]
            wait_s(j + k, rows, ssem, csem)
            start_g(j + k + NB, rows, gsem)

    base = trip - NB
    for k in range(NB):
        rows, gsem, ssem, csem = bufs[k][0], bufs[k][1], sbufs[k], cbufs[k]
        wait_g(base + k, rows, gsem)
        start_s(base + k, rows, ssem, csem)
    for k in range(NB):
        rows, gsem, ssem, csem = bufs[k][0], bufs[k][1], sbufs[k], cbufs[k]
        wait_s(base + k, rows, ssem, csem)

    plsc.subcore_barrier()

    wrows = bufs[0][0]
    for j in range(WB):
        pltpu.sync_copy(acc.at[pl.ds(rbase + j * CH, CH)], wrows)
        pltpu.sync_copy(
            wrows, osum.at[pl.ds(rbase + j * CH, CH), pl.ds(chalf, DH2)])
    if with_counts:
        @pl.when(c == count_core)
        def _():
            for j in range(WB):
                pltpu.sync_copy(cnt.at[pl.ds(rbase + j * CH, CH)], cbuf)
                pltpu.sync_copy(cbuf, ocnt.at[pl.ds(rbase + j * CH, CH)])
    plsc.subcore_barrier()


@functools.partial(
    pl.kernel,
    out_type=[
        jax.ShapeDtypeStruct((R, D_H), jnp.float32),
        jax.ShapeDtypeStruct((R,), jnp.float32),
        jax.ShapeDtypeStruct((R, D_H), jnp.float32),
        jax.ShapeDtypeStruct((R,), jnp.float32),
    ],
    mesh=_mesh,
    scratch_types=(
        [pltpu.VMEM((CPT, CH), jnp.int32)] * 2
        + [pltpu.VMEM((CH, DH2), jnp.float32)] * NB
        + [pltpu.SemaphoreType.DMA] * (3 * NB)
        + [
            pltpu.VMEM((CH,), jnp.float32),
            pltpu.VMEM((CH, DH2), jnp.float32),
            pltpu.VMEM((CH,), jnp.float32),
            pltpu.VMEM((CH,), jnp.float32),
            pltpu.VMEM_SHARED((R, DH2), jnp.float32),
            pltpu.VMEM_SHARED((R,), jnp.float32),
        ]
    ),
    compiler_params=_sc_params,
)
def _sc_seg_both(ha, hp, srcw, dstw, srcc, dstc, z2, z1, o1,
                 osw, ocw, osc, occ,
                 idx_s, idx_d, *scr):
    rows = scr[:NB]
    gsems = scr[NB:2 * NB]
    sbufs = list(scr[2 * NB:3 * NB])
    cbufs = list(scr[3 * NB:4 * NB])
    ones_v, zbuf, zc, cbuf, acc, cnt = scr[4 * NB:]
    c = lax.axis_index("c")
    s = lax.axis_index("s")
    bufs = list(zip(rows, gsems))
    pltpu.sync_copy(z2, zbuf)
    pltpu.sync_copy(z1, zc)
    pltpu.sync_copy(o1, ones_v)
    _seg_pass(ha, srcw, dstw, osw, ocw, 0,
              idx_s, idx_d, bufs, sbufs, cbufs,
              ones_v, zbuf, zc, cbuf, acc, cnt, c, s)
    _seg_pass(hp, srcc, dstc, osc, occ, 1,
              idx_s, idx_d, bufs, sbufs, cbufs,
              ones_v, zbuf, zc, cbuf, acc, cnt, c, s)


@functools.partial(
    pl.kernel,
    out_type=[jax.ShapeDtypeStruct((R, D_H), jnp.float32)],
    mesh=_mesh,
    scratch_types=(
        [pltpu.VMEM((CPT, CH), jnp.int32)] * 2
        + [pltpu.VMEM((CH, DH2), jnp.float32)] * NB
        + [pltpu.SemaphoreType.DMA] * (2 * NB)
        + [
            pltpu.VMEM((CH, DH2), jnp.float32),
            pltpu.VMEM_SHARED((R, DH2), jnp.float32),
        ]
    ),
    compiler_params=_sc_params,
)
def _sc_seg_cites(hp, srcc, dstc, z2, osc,
                  idx_s, idx_d, *scr):
    rows = scr[:NB]
    gsems = scr[NB:2 * NB]
    sbufs = list(scr[2 * NB:3 * NB])
    zbuf, acc = scr[3 * NB:]
    c = lax.axis_index("c")
    s = lax.axis_index("s")
    bufs = list(zip(rows, gsems))
    pltpu.sync_copy(z2, zbuf)
    _seg_pass(hp, srcc, dstc, osc, None, None,
              idx_s, idx_d, bufs, sbufs, [None] * NB,
              None, zbuf, None, None, acc, None, c, s)


def _proj_body(xa_ref, xp_ref, wa_ref, wp_ref, ha_ref, hp_ref):
    ha_ref[...] = jnp.dot(xa_ref[...], wa_ref[...],
                          preferred_element_type=jnp.float32)
    hp_ref[...] = jnp.dot(xp_ref[...], wp_ref[...],
                          preferred_element_type=jnp.float32)


def _project(xa, xp, wa, wp):
    return pl.pallas_call(
        _proj_body,
        grid=(GRID,),
        in_specs=[
            pl.BlockSpec((BM, D_IN), lambda i: (i, 0)),
            pl.BlockSpec((BM, D_IN), lambda i: (i, 0)),
            pl.BlockSpec((D_IN, D_H), lambda i: (0, 0)),
            pl.BlockSpec((D_IN, D_H), lambda i: (0, 0)),
        ],
        out_specs=[
            pl.BlockSpec((BM, D_H), lambda i: (i, 0)),
            pl.BlockSpec((BM, D_H), lambda i: (i, 0)),
        ],
        out_shape=[
            jax.ShapeDtypeStruct((N_NODE, D_H), jnp.float32),
            jax.ShapeDtypeStruct((N_NODE, D_H), jnp.float32),
        ],
    )(xa, xp, wa, wp)


def _inv_counts_body(cw_ref, cc_ref, iw_ref, ic_ref):
    r0 = lax.broadcasted_iota(jnp.int32, (D_H, D_H), 0)
    r1 = lax.broadcasted_iota(jnp.int32, (D_H, D_H), 1)
    sel = (r0 == r1).astype(jnp.float32)
    for src, dst in ((cw_ref, iw_ref), (cc_ref, ic_ref)):
        inv8 = 1.0 / jnp.maximum(src[...], 1.0)
        for k in range(8):
            rb = jnp.broadcast_to(inv8[k:k + 1, :], (D_H, D_H))
            part = lax.dot_general(rb, sel, (((0,), (0,)), ((), ())),
                                   preferred_element_type=jnp.float32)
            dst[pl.ds(k * D_H, D_H), :] = part


def _inv_counts(cw, cc):
    return pl.pallas_call(
        _inv_counts_body,
        grid=(GRID,),
        in_specs=[
            pl.BlockSpec((8, D_H), lambda i: (i, 0)),
            pl.BlockSpec((8, D_H), lambda i: (i, 0)),
        ],
        out_specs=[
            pl.BlockSpec((BM, D_H), lambda i: (i, 0)),
            pl.BlockSpec((BM, D_H), lambda i: (i, 0)),
        ],
        out_shape=[
            jax.ShapeDtypeStruct((R, D_H), jnp.float32),
            jax.ShapeDtypeStruct((R, D_H), jnp.float32),
        ],
    )(cw.reshape(R // D_H, D_H), cc.reshape(R // D_H, D_H))


def _combine_body(sw_ref, iw_ref, sc_ref, ic_ref, hp_ref,
                  wlw_ref, wlc_ref, wrw_ref, wrc_ref, blw_ref, blc_ref,
                  out_ref):
    dot = functools.partial(jnp.dot, preferred_element_type=jnp.float32)
    acc = dot(sw_ref[...] * iw_ref[...], wlw_ref[...])
    acc = acc + dot(sc_ref[...] * ic_ref[...], wlc_ref[...])
    acc = acc + dot(hp_ref[...], wrw_ref[...] + wrc_ref[...])
    acc = acc + blw_ref[...] + blc_ref[...]
    out_ref[...] = jnp.maximum(acc * 0.5, 0.0)


def _combine(sw, iw, sc, ic, hp, wlw, wlc, wrw, wrc, blw, blc):
    return pl.pallas_call(
        _combine_body,
        grid=(GRID,),
        in_specs=[
            pl.BlockSpec((BM, D_H), lambda i: (i, 0)),
            pl.BlockSpec((BM, D_H), lambda i: (i, 0)),
            pl.BlockSpec((BM, D_H), lambda i: (i, 0)),
            pl.BlockSpec((BM, D_H), lambda i: (i, 0)),
            pl.BlockSpec((BM, D_H), lambda i: (i, 0)),
            pl.BlockSpec((D_H, D_H), lambda i: (0, 0)),
            pl.BlockSpec((D_H, D_H), lambda i: (0, 0)),
            pl.BlockSpec((D_H, D_H), lambda i: (0, 0)),
            pl.BlockSpec((D_H, D_H), lambda i: (0, 0)),
            pl.BlockSpec((1, D_H), lambda i: (0, 0)),
            pl.BlockSpec((1, D_H), lambda i: (0, 0)),
        ],
        out_specs=pl.BlockSpec((BM, D_H), lambda i: (i, 0)),
        out_shape=jax.ShapeDtypeStruct((N_NODE, D_H), jnp.float32),
    )(sw, iw, sc, ic, hp, wlw, wlc, wrw, wrc, blw, blc)


def _prep_edges(ei):
    pad = EP - E
    src = jnp.concatenate([2 * ei[0], jnp.zeros((pad,), jnp.int32)])
    dst = jnp.concatenate([ei[1], jnp.full((pad,), N_NODE, jnp.int32)])
    return src.reshape(NS, CPT, CH), dst.reshape(NS, CPT, CH)


def kernel(x_author, x_paper, edge_index_writes, edge_index_cites,
           W_in_author, W_in_paper,
           Wl_w0, bl_w0, Wr_w0, Wl_c0, bl_c0, Wr_c0,
           Wl_w1, bl_w1, Wr_w1, Wl_c1, bl_c1, Wr_c1):
    srcw, dstw = _prep_edges(edge_index_writes)
    srcc, dstc = _prep_edges(edge_index_cites)
    z2 = jnp.zeros((CH, DH2), jnp.float32)
    z1 = jnp.zeros((CH,), jnp.float32)
    o1 = jnp.ones((CH,), jnp.float32)

    haf, hp0 = _project(x_author, x_paper, W_in_author, W_in_paper)

    sw, cw, sc0, cc = _sc_seg_both(haf.reshape(2 * N_NODE, DH2),
                                   hp0.reshape(2 * N_NODE, DH2),
                                   srcw, dstw, srcc, dstc, z2, z1, o1)
    iw, ic = _inv_counts(cw, cc)

    blw0 = bl_w0.reshape(1, D_H)
    blc0 = bl_c0.reshape(1, D_H)
    hp1 = _combine(sw, iw, sc0, ic, hp0,
                   Wl_w0, Wl_c0, Wr_w0, Wr_c0, blw0, blc0)

    (sc1,) = _sc_seg_cites(hp1.reshape(2 * N_NODE, DH2), srcc, dstc, z2)

    blw1 = bl_w1.reshape(1, D_H)
    blc1 = bl_c1.reshape(1, D_H)
    hp2 = _combine(sw, iw, sc1, ic, hp1,
                   Wl_w1, Wl_c1, Wr_w1, Wr_c1, blw1, blc1)

    return (haf, hp2)


# speedup vs baseline: 12.6460x; 1.0515x over previous
import functools

import jax
import jax.numpy as jnp
from jax import lax
from jax.experimental import pallas as pl
from jax.experimental.pallas import tpu as pltpu
from jax.experimental.pallas import tpu_sc as plsc

N_NODE = 10000
D_IN = 256
D_H = 128
DH2 = D_H // 2
E = 320000

NC = 2
NS = 16
CH = 128
CPT = 160
EP = NS * CPT * CH
NCH_REAL = E // CH
CPT_LAST = NCH_REAL - (NS - 1) * CPT
R = 10240
RPT = R // NS
WB = RPT // CH
NB = 4
BM = 1024
GRID = R // BM

_mesh = plsc.VectorSubcoreMesh(core_axis_name="c", subcore_axis_name="s")
_sc_params = pltpu.CompilerParams(use_tc_tiling_on_sc=False)


def _seg_pass(h_hbm, src_hbm, dst_hbm, osum, ocnt, count_core,
              idx_s, idx_d, bufs, sbufs, cbufs,
              ones_v, zbuf, zc, cbuf, acc, cnt, c, s):
    rbase = s * RPT
    chalf = c * DH2
    with_counts = ocnt is not None

    for j in range(WB):
        pltpu.sync_copy(zbuf, acc.at[pl.ds(rbase + j * CH, CH)])
    if with_counts:
        for j in range(WB):
            pltpu.sync_copy(zc, cnt.at[pl.ds(rbase + j * CH, CH)])
    plsc.subcore_barrier()

    pltpu.sync_copy(src_hbm.at[s], idx_s)
    pltpu.sync_copy(dst_hbm.at[s], idx_d)

    trip = jnp.where(s == NS - 1, CPT_LAST, CPT)

    h_half = h_hbm.at[pl.ds(c, 2 * N_NODE - 1)]

    def start_g(j, rows, sem):
        pltpu.async_copy(h_half.at[idx_s.at[j]], rows, sem)

    def wait_g(j, rows, sem):
        pltpu.make_async_copy(h_half.at[idx_s.at[j]], rows, sem).wait()

    def start_s(j, rows, sem, csem):
        pltpu.async_copy(rows, acc.at[idx_d.at[j]], sem, add=True)
        if with_counts:
            @pl.when(c == count_core)
            def _():
                pltpu.async_copy(ones_v, cnt.at[idx_d.at[j]], csem, add=True)

    def wait_s(j, rows, sem, csem):
        pltpu.make_async_copy(rows, acc.at[idx_d.at[j]], sem).wait()
        if with_counts:
            @pl.when(c == count_core)
            def _():
                pltpu.make_async_copy(ones_v, cnt.at[idx_d.at[j]], csem).wait()

    for k in range(NB):
        start_g(k, *bufs[k])

    @pl.loop(0, trip - NB, step=NB)
    def _(j):
        for k in range(NB):
            rows, gsem, ssem, csem = bufs[k][0], bufs[k][1], sbufs[k], cbufs[k]
            wait_g(j + k, rows, gsem)
            start_s(j + k, rows, ssem, csem)
        for k in range(NB):
            rows, gsem, ssem, csem = bufs[k][0], bufs[k][1], sbufs[k], cbufs[k]
            wait_s(j + k, rows, ssem, csem)
            start_g(j + k + NB, rows, gsem)

    base = trip - NB
    for k in range(NB):
        rows, gsem, ssem, csem = bufs[k][0], bufs[k][1], sbufs[k], cbufs[k]
        wait_g(base + k, rows, gsem)
        start_s(base + k, rows, ssem, csem)
    for k in range(NB):
        rows, gsem, ssem, csem = bufs[k][0], bufs[k][1], sbufs[k], cbufs[k]
        wait_s(base + k, rows, ssem, csem)

    plsc.subcore_barrier()

    wrows = bufs[0][0]
    for j in range(WB):
        pltpu.sync_copy(acc.at[pl.ds(rbase + j * CH, CH)], wrows)
        pltpu.sync_copy(
            wrows, osum.at[pl.ds(rbase + j * CH, CH), pl.ds(chalf, DH2)])
    if with_counts:
        @pl.when(c == count_core)
        def _():
            for j in range(WB):
                pltpu.sync_copy(cnt.at[pl.ds(rbase + j * CH, CH)], cbuf)
                pltpu.sync_copy(cbuf, ocnt.at[pl.ds(rbase + j * CH, CH)])
    plsc.subcore_barrier()


@functools.partial(
    pl.kernel,
    out_type=[
        jax.ShapeDtypeStruct((R, D_H), jnp.float32),
        jax.ShapeDtypeStruct((R,), jnp.float32),
        jax.ShapeDtypeStruct((R, D_H), jnp.float32),
        jax.ShapeDtypeStruct((R,), jnp.float32),
    ],
    mesh=_mesh,
    scratch_types=(
        [pltpu.VMEM((CPT, CH), jnp.int32)] * 2
        + [pltpu.VMEM((CH, DH2), jnp.float32)] * NB
        + [pltpu.SemaphoreType.DMA] * (3 * NB)
        + [
            pltpu.VMEM((CH,), jnp.float32),
            pltpu.VMEM((CH, DH2), jnp.float32),
            pltpu.VMEM((CH,), jnp.float32),
            pltpu.VMEM((CH,), jnp.float32),
            pltpu.VMEM_SHARED((R, DH2), jnp.float32),
            pltpu.VMEM_SHARED((R,), jnp.float32),
        ]
    ),
    compiler_params=_sc_params,
)
def _sc_seg_both(ha, hp, srcw, dstw, srcc, dstc, z2, z1, o1,
                 osw, ocw, osc, occ,
                 idx_s, idx_d, *scr):
    rows = scr[:NB]
    gsems = scr[NB:2 * NB]
    sbufs = list(scr[2 * NB:3 * NB])
    cbufs = list(scr[3 * NB:4 * NB])
    ones_v, zbuf, zc, cbuf, acc, cnt = scr[4 * NB:]
    c = lax.axis_index("c")
    s = lax.axis_index("s")
    bufs = list(zip(rows, gsems))
    pltpu.sync_copy(z2, zbuf)
    pltpu.sync_copy(z1, zc)
    pltpu.sync_copy(o1, ones_v)
    _seg_pass(ha, srcw, dstw, osw, ocw, 0,
              idx_s, idx_d, bufs, sbufs, cbufs,
              ones_v, zbuf, zc, cbuf, acc, cnt, c, s)
    _seg_pass(hp, srcc, dstc, osc, occ, 1,
              idx_s, idx_d, bufs, sbufs, cbufs,
              ones_v, zbuf, zc, cbuf, acc, cnt, c, s)


@functools.partial(
    pl.kernel,
    out_type=[jax.ShapeDtypeStruct((R, D_H), jnp.float32)],
    mesh=_mesh,
    scratch_types=(
        [pltpu.VMEM((CPT, CH), jnp.int32)] * 2
        + [pltpu.VMEM((CH, DH2), jnp.float32)] * NB
        + [pltpu.SemaphoreType.DMA] * (2 * NB)
        + [
            pltpu.VMEM((CH, DH2), jnp.float32),
            pltpu.VMEM_SHARED((R, DH2), jnp.float32),
        ]
    ),
    compiler_params=_sc_params,
)
def _sc_seg_cites(hp, srcc, dstc, z2, osc,
                  idx_s, idx_d, *scr):
    rows = scr[:NB]
    gsems = scr[NB:2 * NB]
    sbufs = list(scr[2 * NB:3 * NB])
    zbuf, acc = scr[3 * NB:]
    c = lax.axis_index("c")
    s = lax.axis_index("s")
    bufs = list(zip(rows, gsems))
    pltpu.sync_copy(z2, zbuf)
    _seg_pass(hp, srcc, dstc, osc, None, None,
              idx_s, idx_d, bufs, sbufs, [None] * NB,
              None, zbuf, None, None, acc, None, c, s)


def _proj_body(xa_ref, xp_ref, wa_ref, wp_ref, ha_ref, hp_ref):
    ha_ref[...] = jnp.dot(xa_ref[...], wa_ref[...],
                          preferred_element_type=jnp.float32)
    hp_ref[...] = jnp.dot(xp_ref[...], wp_ref[...],
                          preferred_element_type=jnp.float32)


def _project(xa, xp, wa, wp):
    return pl.pallas_call(
        _proj_body,
        grid=(GRID,),
        in_specs=[
            pl.BlockSpec((BM, D_IN), lambda i: (i, 0)),
            pl.BlockSpec((BM, D_IN), lambda i: (i, 0)),
            pl.BlockSpec((D_IN, D_H), lambda i: (0, 0)),
            pl.BlockSpec((D_IN, D_H), lambda i: (0, 0)),
        ],
        out_specs=[
            pl.BlockSpec((BM, D_H), lambda i: (i, 0)),
            pl.BlockSpec((BM, D_H), lambda i: (i, 0)),
        ],
        out_shape=[
            jax.ShapeDtypeStruct((N_NODE, D_H), jnp.float32),
            jax.ShapeDtypeStruct((N_NODE, D_H), jnp.float32),
        ],
    )(xa, xp, wa, wp)


def _inv_counts_body(cw_ref, cc_ref, iw_ref, ic_ref):
    r0 = lax.broadcasted_iota(jnp.int32, (D_H, D_H), 0)
    r1 = lax.broadcasted_iota(jnp.int32, (D_H, D_H), 1)
    sel = (r0 == r1).astype(jnp.float32)
    for src, dst in ((cw_ref, iw_ref), (cc_ref, ic_ref)):
        inv8 = 1.0 / jnp.maximum(src[...], 1.0)
        for k in range(8):
            rb = jnp.broadcast_to(inv8[k:k + 1, :], (D_H, D_H))
            part = lax.dot_general(rb, sel, (((0,), (0,)), ((), ())),
                                   preferred_element_type=jnp.float32)
            dst[pl.ds(k * D_H, D_H), :] = part


def _inv_counts(cw, cc):
    return pl.pallas_call(
        _inv_counts_body,
        grid=(GRID,),
        in_specs=[
            pl.BlockSpec((8, D_H), lambda i: (i, 0)),
            pl.BlockSpec((8, D_H), lambda i: (i, 0)),
        ],
        out_specs=[
            pl.BlockSpec((BM, D_H), lambda i: (i, 0)),
            pl.BlockSpec((BM, D_H), lambda i: (i, 0)),
        ],
        out_shape=[
            jax.ShapeDtypeStruct((R, D_H), jnp.float32),
            jax.ShapeDtypeStruct((R, D_H), jnp.float32),
        ],
    )(cw.reshape(R // D_H, D_H), cc.reshape(R // D_H, D_H))


def _combine_body(sw_ref, iw_ref, sc_ref, ic_ref, hp_ref,
                  wlw_ref, wlc_ref, wrw_ref, wrc_ref, blw_ref, blc_ref,
                  out_ref):
    dot = functools.partial(jnp.dot, preferred_element_type=jnp.float32)
    acc = dot(sw_ref[...] * iw_ref[...], wlw_ref[...])
    acc = acc + dot(sc_ref[...] * ic_ref[...], wlc_ref[...])
    acc = acc + dot(hp_ref[...], wrw_ref[...] + wrc_ref[...])
    acc = acc + blw_ref[...] + blc_ref[...]
    out_ref[...] = jnp.maximum(acc * 0.5, 0.0)


def _combine(sw, iw, sc, ic, hp, wlw, wlc, wrw, wrc, blw, blc):
    return pl.pallas_call(
        _combine_body,
        grid=(GRID,),
        in_specs=[
            pl.BlockSpec((BM, D_H), lambda i: (i, 0)),
            pl.BlockSpec((BM, D_H), lambda i: (i, 0)),
            pl.BlockSpec((BM, D_H), lambda i: (i, 0)),
            pl.BlockSpec((BM, D_H), lambda i: (i, 0)),
            pl.BlockSpec((BM, D_H), lambda i: (i, 0)),
            pl.BlockSpec((D_H, D_H), lambda i: (0, 0)),
            pl.BlockSpec((D_H, D_H), lambda i: (0, 0)),
            pl.BlockSpec((D_H, D_H), lambda i: (0, 0)),
            pl.BlockSpec((D_H, D_H), lambda i: (0, 0)),
            pl.BlockSpec((1, D_H), lambda i: (0, 0)),
            pl.BlockSpec((1, D_H), lambda i: (0, 0)),
        ],
        out_specs=pl.BlockSpec((BM, D_H), lambda i: (i, 0)),
        out_shape=jax.ShapeDtypeStruct((N_NODE, D_H), jnp.float32),
    )(sw, iw, sc, ic, hp, wlw, wlc, wrw, wrc, blw, blc)


EPT = CPT * CH


def _edges_body(ew_ref, ec_ref, sw_ref, dw_ref, sc_ref, dc_ref):
    for ei_ref, s_ref, d_ref in ((ew_ref, sw_ref, dw_ref),
                                 (ec_ref, sc_ref, dc_ref)):
        for k in range(CPT):
            blk = ei_ref[:, pl.ds(k * CH, CH)]
            s_ref[0, k, :] = 2 * blk[0]
            d_ref[0, k, :] = blk[1]


def _prep_edges(ew, ec):
    out = pl.pallas_call(
        _edges_body,
        grid=(NS,),
        in_specs=[
            pl.BlockSpec((2, EPT), lambda i: (0, i)),
            pl.BlockSpec((2, EPT), lambda i: (0, i)),
        ],
        out_specs=[pl.BlockSpec((1, CPT, CH), lambda i: (i, 0, 0))] * 4,
        out_shape=[jax.ShapeDtypeStruct((NS, CPT, CH), jnp.int32)] * 4,
    )(ew, ec)
    return out


def kernel(x_author, x_paper, edge_index_writes, edge_index_cites,
           W_in_author, W_in_paper,
           Wl_w0, bl_w0, Wr_w0, Wl_c0, bl_c0, Wr_c0,
           Wl_w1, bl_w1, Wr_w1, Wl_c1, bl_c1, Wr_c1):
    srcw, dstw, srcc, dstc = _prep_edges(edge_index_writes, edge_index_cites)
    z2 = jnp.zeros((CH, DH2), jnp.float32)
    z1 = jnp.zeros((CH,), jnp.float32)
    o1 = jnp.ones((CH,), jnp.float32)

    haf, hp0 = _project(x_author, x_paper, W_in_author, W_in_paper)

    sw, cw, sc0, cc = _sc_seg_both(haf.reshape(2 * N_NODE, DH2),
                                   hp0.reshape(2 * N_NODE, DH2),
                                   srcw, dstw, srcc, dstc, z2, z1, o1)
    iw, ic = _inv_counts(cw, cc)

    blw0 = bl_w0.reshape(1, D_H)
    blc0 = bl_c0.reshape(1, D_H)
    hp1 = _combine(sw, iw, sc0, ic, hp0,
                   Wl_w0, Wl_c0, Wr_w0, Wr_c0, blw0, blc0)

    (sc1,) = _sc_seg_cites(hp1.reshape(2 * N_NODE, DH2), srcc, dstc, z2)

    blw1 = bl_w1.reshape(1, D_H)
    blc1 = bl_c1.reshape(1, D_H)
    hp2 = _combine(sw, iw, sc1, ic, hp1,
                   Wl_w1, Wl_c1, Wr_w1, Wr_c1, blw1, blc1)

    return (haf, hp2)


# speedup vs baseline: 12.6741x; 1.0022x over previous
import functools

import jax
import jax.numpy as jnp
from jax import lax
from jax.experimental import pallas as pl
from jax.experimental.pallas import tpu as pltpu
from jax.experimental.pallas import tpu_sc as plsc

N_NODE = 10000
D_IN = 256
D_H = 128
DH2 = D_H // 2
E = 320000

NC = 2
NS = 16
CH = 128
CPT = 160
EP = NS * CPT * CH
NCH_REAL = E // CH
CPT_LAST = NCH_REAL - (NS - 1) * CPT
R = 10240
RPT = R // NS
WB = RPT // CH
NB = 4
BM = 1024
GRID = R // BM

_mesh = plsc.VectorSubcoreMesh(core_axis_name="c", subcore_axis_name="s")
_sc_params = pltpu.CompilerParams(use_tc_tiling_on_sc=False)


def _seg_pass(h_hbm, src_hbm, dst_hbm, osum, ocnt, count_core,
              idx_s, idx_d, bufs, sbufs, cbufs,
              ones_v, zbuf, zc, cbuf, acc, cnt, c, s):
    rbase = s * RPT
    chalf = c * DH2
    with_counts = ocnt is not None

    for j in range(WB):
        pltpu.sync_copy(zbuf, acc.at[pl.ds(rbase + j * CH, CH)])
    if with_counts:
        for j in range(WB):
            pltpu.sync_copy(zc, cnt.at[pl.ds(rbase + j * CH, CH)])
    plsc.subcore_barrier()

    pltpu.sync_copy(src_hbm.at[s], idx_s)
    pltpu.sync_copy(dst_hbm.at[s], idx_d)

    trip = jnp.where(s == NS - 1, CPT_LAST, CPT)

    h_half = h_hbm.at[pl.ds(c, 2 * N_NODE - 1)]

    def start_g(j, rows, sem):
        pltpu.async_copy(h_half.at[idx_s.at[j]], rows, sem)

    def wait_g(j, rows, sem):
        pltpu.make_async_copy(h_half.at[idx_s.at[j]], rows, sem).wait()

    def start_s(j, rows, sem, csem):
        pltpu.async_copy(rows, acc.at[idx_d.at[j]], sem, add=True)
        if with_counts:
            @pl.when(c == count_core)
            def _():
                pltpu.async_copy(ones_v, cnt.at[idx_d.at[j]], csem, add=True)

    def wait_s(j, rows, sem, csem):
        pltpu.make_async_copy(rows, acc.at[idx_d.at[j]], sem).wait()
        if with_counts:
            @pl.when(c == count_core)
            def _():
                pltpu.make_async_copy(ones_v, cnt.at[idx_d.at[j]], csem).wait()

    for k in range(NB):
        start_g(k, *bufs[k])

    @pl.loop(0, trip - NB, step=NB)
    def _(j):
        for k in range(NB):
            rows, gsem, ssem, csem = bufs[k][0], bufs[k][1], sbufs[k], cbufs[k]
            wait_g(j + k, rows, gsem)
            start_s(j + k, rows, ssem, csem)
        for k in range(NB):
            rows, gsem, ssem, csem = bufs[k][0], bufs[k][1], sbufs[k], cbufs[k]
            wait_s(j + k, rows, ssem, csem)
            start_g(j + k + NB, rows, gsem)

    base = trip - NB
    for k in range(NB):
        rows, gsem, ssem, csem = bufs[k][0], bufs[k][1], sbufs[k], cbufs[k]
        wait_g(base + k, rows, gsem)
        start_s(base + k, rows, ssem, csem)
    for k in range(NB):
        rows, gsem, ssem, csem = bufs[k][0], bufs[k][1], sbufs[k], cbufs[k]
        wait_s(base + k, rows, ssem, csem)

    plsc.subcore_barrier()

    wrows = bufs[0][0]
    for j in range(WB):
        pltpu.sync_copy(acc.at[pl.ds(rbase + j * CH, CH)], wrows)
        pltpu.sync_copy(
            wrows, osum.at[pl.ds(rbase + j * CH, CH), pl.ds(chalf, DH2)])
    if with_counts:
        @pl.when(c == count_core)
        def _():
            for j in range(WB):
                pltpu.sync_copy(cnt.at[pl.ds(rbase + j * CH, CH)], cbuf)
                pltpu.sync_copy(cbuf, ocnt.at[pl.ds(rbase + j * CH, CH)])
    plsc.subcore_barrier()


@functools.partial(
    pl.kernel,
    out_type=[
        jax.ShapeDtypeStruct((R, D_H), jnp.float32),
        jax.ShapeDtypeStruct((R,), jnp.float32),
        jax.ShapeDtypeStruct((R, D_H), jnp.float32),
        jax.ShapeDtypeStruct((R,), jnp.float32),
    ],
    mesh=_mesh,
    scratch_types=(
        [pltpu.VMEM((CPT, CH), jnp.int32)] * 2
        + [pltpu.VMEM((CH, DH2), jnp.float32)] * NB
        + [pltpu.SemaphoreType.DMA] * (3 * NB)
        + [
            pltpu.VMEM((CH,), jnp.float32),
            pltpu.VMEM((CH, DH2), jnp.float32),
            pltpu.VMEM((CH,), jnp.float32),
            pltpu.VMEM((CH,), jnp.float32),
            pltpu.VMEM_SHARED((R, DH2), jnp.float32),
            pltpu.VMEM_SHARED((R,), jnp.float32),
        ]
    ),
    compiler_params=_sc_params,
)
def _sc_seg_both(ha, hp, srcw, dstw, srcc, dstc, z2, z1, o1,
                 osw, ocw, osc, occ,
                 idx_s, idx_d, *scr):
    rows = scr[:NB]
    gsems = scr[NB:2 * NB]
    sbufs = list(scr[2 * NB:3 * NB])
    cbufs = list(scr[3 * NB:4 * NB])
    ones_v, zbuf, zc, cbuf, acc, cnt = scr[4 * NB:]
    c = lax.axis_index("c")
    s = lax.axis_index("s")
    bufs = list(zip(rows, gsems))
    pltpu.sync_copy(z2, zbuf)
    pltpu.sync_copy(z1, zc)
    pltpu.sync_copy(o1, ones_v)
    _seg_pass(ha, srcw, dstw, osw, ocw, 0,
              idx_s, idx_d, bufs, sbufs, cbufs,
              ones_v, zbuf, zc, cbuf, acc, cnt, c, s)
    _seg_pass(hp, srcc, dstc, osc, occ, 1,
              idx_s, idx_d, bufs, sbufs, cbufs,
              ones_v, zbuf, zc, cbuf, acc, cnt, c, s)


@functools.partial(
    pl.kernel,
    out_type=[jax.ShapeDtypeStruct((R, D_H), jnp.float32)],
    mesh=_mesh,
    scratch_types=(
        [pltpu.VMEM((CPT, CH), jnp.int32)] * 2
        + [pltpu.VMEM((CH, DH2), jnp.float32)] * NB
        + [pltpu.SemaphoreType.DMA] * (2 * NB)
        + [
            pltpu.VMEM((CH, DH2), jnp.float32),
            pltpu.VMEM_SHARED((R, DH2), jnp.float32),
        ]
    ),
    compiler_params=_sc_params,
)
def _sc_seg_cites(hp, srcc, dstc, z2, osc,
                  idx_s, idx_d, *scr):
    rows = scr[:NB]
    gsems = scr[NB:2 * NB]
    sbufs = list(scr[2 * NB:3 * NB])
    zbuf, acc = scr[3 * NB:]
    c = lax.axis_index("c")
    s = lax.axis_index("s")
    bufs = list(zip(rows, gsems))
    pltpu.sync_copy(z2, zbuf)
    _seg_pass(hp, srcc, dstc, osc, None, None,
              idx_s, idx_d, bufs, sbufs, [None] * NB,
              None, zbuf, None, None, acc, None, c, s)


_DOT = functools.partial(jnp.dot, preferred_element_type=jnp.float32,
                         precision=jax.lax.Precision.DEFAULT)


def _proj_body(xa_ref, xp_ref, wa_ref, wp_ref, ha_ref, hp_ref):
    ha_ref[...] = _DOT(xa_ref[...], wa_ref[...])
    hp_ref[...] = _DOT(xp_ref[...], wp_ref[...])


def _project(xa, xp, wa, wp):
    return pl.pallas_call(
        _proj_body,
        grid=(GRID,),
        in_specs=[
            pl.BlockSpec((BM, D_IN), lambda i: (i, 0)),
            pl.BlockSpec((BM, D_IN), lambda i: (i, 0)),
            pl.BlockSpec((D_IN, D_H), lambda i: (0, 0)),
            pl.BlockSpec((D_IN, D_H), lambda i: (0, 0)),
        ],
        out_specs=[
            pl.BlockSpec((BM, D_H), lambda i: (i, 0)),
            pl.BlockSpec((BM, D_H), lambda i: (i, 0)),
        ],
        out_shape=[
            jax.ShapeDtypeStruct((N_NODE, D_H), jnp.float32),
            jax.ShapeDtypeStruct((N_NODE, D_H), jnp.float32),
        ],
    )(xa, xp, wa, wp)


def _inv_counts_body(cw_ref, cc_ref, iw_ref, ic_ref):
    r0 = lax.broadcasted_iota(jnp.int32, (D_H, D_H), 0)
    r1 = lax.broadcasted_iota(jnp.int32, (D_H, D_H), 1)
    sel = (r0 == r1).astype(jnp.float32)
    for src, dst in ((cw_ref, iw_ref), (cc_ref, ic_ref)):
        inv8 = 1.0 / jnp.maximum(src[...], 1.0)
        for k in range(8):
            rb = jnp.broadcast_to(inv8[k:k + 1, :], (D_H, D_H))
            part = lax.dot_general(rb, sel, (((0,), (0,)), ((), ())),
                                   preferred_element_type=jnp.float32)
            dst[pl.ds(k * D_H, D_H), :] = part


def _inv_counts(cw, cc):
    return pl.pallas_call(
        _inv_counts_body,
        grid=(GRID,),
        in_specs=[
            pl.BlockSpec((8, D_H), lambda i: (i, 0)),
            pl.BlockSpec((8, D_H), lambda i: (i, 0)),
        ],
        out_specs=[
            pl.BlockSpec((BM, D_H), lambda i: (i, 0)),
            pl.BlockSpec((BM, D_H), lambda i: (i, 0)),
        ],
        out_shape=[
            jax.ShapeDtypeStruct((R, D_H), jnp.float32),
            jax.ShapeDtypeStruct((R, D_H), jnp.float32),
        ],
    )(cw.reshape(R // D_H, D_H), cc.reshape(R // D_H, D_H))


def _combine_body(sw_ref, iw_ref, sc_ref, ic_ref, hp_ref,
                  wlw_ref, wlc_ref, wrw_ref, wrc_ref, blw_ref, blc_ref,
                  out_ref):
    acc = _DOT(sw_ref[...] * iw_ref[...], wlw_ref[...])
    acc = acc + _DOT(sc_ref[...] * ic_ref[...], wlc_ref[...])
    acc = acc + _DOT(hp_ref[...], wrw_ref[...] + wrc_ref[...])
    acc = acc + blw_ref[...] + blc_ref[...]
    out_ref[...] = jnp.maximum(acc * 0.5, 0.0)


def _combine(sw, iw, sc, ic, hp, wlw, wlc, wrw, wrc, blw, blc):
    return pl.pallas_call(
        _combine_body,
        grid=(GRID,),
        in_specs=[
            pl.BlockSpec((BM, D_H), lambda i: (i, 0)),
            pl.BlockSpec((BM, D_H), lambda i: (i, 0)),
            pl.BlockSpec((BM, D_H), lambda i: (i, 0)),
            pl.BlockSpec((BM, D_H), lambda i: (i, 0)),
            pl.BlockSpec((BM, D_H), lambda i: (i, 0)),
            pl.BlockSpec((D_H, D_H), lambda i: (0, 0)),
            pl.BlockSpec((D_H, D_H), lambda i: (0, 0)),
            pl.BlockSpec((D_H, D_H), lambda i: (0, 0)),
            pl.BlockSpec((D_H, D_H), lambda i: (0, 0)),
            pl.BlockSpec((1, D_H), lambda i: (0, 0)),
            pl.BlockSpec((1, D_H), lambda i: (0, 0)),
        ],
        out_specs=pl.BlockSpec((BM, D_H), lambda i: (i, 0)),
        out_shape=jax.ShapeDtypeStruct((N_NODE, D_H), jnp.float32),
    )(sw, iw, sc, ic, hp, wlw, wlc, wrw, wrc, blw, blc)


EPT = CPT * CH


def _edges_body(ew_ref, ec_ref, sw_ref, dw_ref, sc_ref, dc_ref):
    for ei_ref, s_ref, d_ref in ((ew_ref, sw_ref, dw_ref),
                                 (ec_ref, sc_ref, dc_ref)):
        for k in range(CPT):
            blk = ei_ref[:, pl.ds(k * CH, CH)]
            s_ref[0, k, :] = 2 * blk[0]
            d_ref[0, k, :] = blk[1]


def _prep_edges(ew, ec):
    out = pl.pallas_call(
        _edges_body,
        grid=(NS,),
        in_specs=[
            pl.BlockSpec((2, EPT), lambda i: (0, i)),
            pl.BlockSpec((2, EPT), lambda i: (0, i)),
        ],
        out_specs=[pl.BlockSpec((1, CPT, CH), lambda i: (i, 0, 0))] * 4,
        out_shape=[jax.ShapeDtypeStruct((NS, CPT, CH), jnp.int32)] * 4,
    )(ew, ec)
    return out


def kernel(x_author, x_paper, edge_index_writes, edge_index_cites,
           W_in_author, W_in_paper,
           Wl_w0, bl_w0, Wr_w0, Wl_c0, bl_c0, Wr_c0,
           Wl_w1, bl_w1, Wr_w1, Wl_c1, bl_c1, Wr_c1):
    srcw, dstw, srcc, dstc = _prep_edges(edge_index_writes, edge_index_cites)
    z2 = jnp.zeros((CH, DH2), jnp.float32)
    z1 = jnp.zeros((CH,), jnp.float32)
    o1 = jnp.ones((CH,), jnp.float32)

    haf, hp0 = _project(x_author, x_paper, W_in_author, W_in_paper)

    sw, cw, sc0, cc = _sc_seg_both(haf.reshape(2 * N_NODE, DH2),
                                   hp0.reshape(2 * N_NODE, DH2),
                                   srcw, dstw, srcc, dstc, z2, z1, o1)
    iw, ic = _inv_counts(cw, cc)

    blw0 = bl_w0.reshape(1, D_H)
    blc0 = bl_c0.reshape(1, D_H)
    hp1 = _combine(sw, iw, sc0, ic, hp0,
                   Wl_w0, Wl_c0, Wr_w0, Wr_c0, blw0, blc0)

    (sc1,) = _sc_seg_cites(hp1.reshape(2 * N_NODE, DH2), srcc, dstc, z2)

    blw1 = bl_w1.reshape(1, D_H)
    blc1 = bl_c1.reshape(1, D_H)
    hp2 = _combine(sw, iw, sc1, ic, hp1,
                   Wl_w1, Wl_c1, Wr_w1, Wr_c1, blw1, blc1)

    return (haf, hp2)
